# baseline (speedup 1.0000x reference)
import functools

import jax
import jax.numpy as jnp
from jax import lax
from jax.experimental import pallas as pl
from jax.experimental.pallas import tpu as pltpu
from jax.experimental.pallas import tpu_sc as plsc

_N = 10000
_E = 320000
_F = 128
_D = 16
_G = 64
_C = 16
_EPS = 1e-5

_NC = 2
_NS = 16
_NW = _NC * _NS
_PER_W = _E // _NW
_B = 40
_CHUNKS = _PER_W // _B
_ROWS_W = 624
_TAIL = _N - _NS * _ROWS_W
_ZROWS = 48

_Q = (0.9999999992732383, -0.49999981736734733, 0.33332569426223596,
      -0.24987394966920914, 0.19891548439724996, -0.16108664201878709,
      0.12425161741270922, -0.08253599813002745, 0.04155807546752951,
      -0.013444998519934246, 0.0020377159799453265)

_PREC = lax.Precision.HIGHEST


def _dot(a, b):
    return jnp.dot(a, b, preferred_element_type=jnp.float32, precision=_PREC)


def _edge_terms_body(ea_ref, w0_ref, b0_ref, w1_ref, b1_ref, o0_ref, o1_ref):
    ea = ea_ref[...]
    o0_ref[...] = _dot(ea, w0_ref[...]) + b0_ref[...]
    o1_ref[...] = _dot(ea, w1_ref[...]) + b1_ref[...]


def _edge_terms(edge_attr, We0, b0, We1, b1):
    eb = 4000
    return pl.pallas_call(
        _edge_terms_body,
        grid=(_E // eb,),
        in_specs=[
            pl.BlockSpec((eb, _D), lambda i: (i, 0)),
            pl.BlockSpec((_D, 2 * _F), lambda i: (0, 0)),
            pl.BlockSpec((1, 2 * _F), lambda i: (0, 0)),
            pl.BlockSpec((_D, 2 * _F), lambda i: (0, 0)),
            pl.BlockSpec((1, 2 * _F), lambda i: (0, 0)),
        ],
        out_specs=[
            pl.BlockSpec((eb, 2 * _F), lambda i: (i, 0)),
            pl.BlockSpec((eb, 2 * _F), lambda i: (i, 0)),
        ],
        out_shape=[jax.ShapeDtypeStruct((_E, 2 * _F), jnp.float32)] * 2,
    )(edge_attr, We0, b0.reshape(1, -1), We1, b1.reshape(1, -1))


def _tables_body(x_ref, wd_ref, ws_ref, td_ref, ts_ref):
    xv = x_ref[...]
    td_ref[...] = _dot(xv, wd_ref[...])
    ts_ref[...] = _dot(xv, ws_ref[...])


def _tables(h, Wd, Wsrc):
    return pl.pallas_call(
        _tables_body,
        out_shape=[jax.ShapeDtypeStruct((_N, 2 * _F), jnp.float32)] * 2,
    )(h, Wd, Wsrc)


_RB = 2000
_NRB = _N // _RB


def _stats_body(x_ref, p_ref, u_ref, sum_ref, sq_ref):
    u = x_ref[...] + p_ref[0] + p_ref[1]
    u_ref[...] = u
    ps = jnp.sum(u, axis=0, keepdims=True)
    pq = jnp.sum(u * u, axis=0, keepdims=True)

    @pl.when(pl.program_id(0) == 0)
    def _init():
        sum_ref[...] = ps
        sq_ref[...] = pq

    @pl.when(pl.program_id(0) > 0)
    def _acc():
        sum_ref[...] += ps
        sq_ref[...] += pq


def _stats(x, parts):
    return pl.pallas_call(
        _stats_body,
        grid=(_NRB,),
        in_specs=[
            pl.BlockSpec((_RB, _F), lambda i: (i, 0)),
            pl.BlockSpec((2, _RB, _F), lambda i: (0, i, 0)),
        ],
        out_specs=[
            pl.BlockSpec((_RB, _F), lambda i: (i, 0)),
            pl.BlockSpec((1, _F), lambda i: (0, 0)),
            pl.BlockSpec((1, _F), lambda i: (0, 0)),
        ],
        out_shape=[
            jax.ShapeDtypeStruct((_N, _F), jnp.float32),
            jax.ShapeDtypeStruct((1, _F), jnp.float32),
            jax.ShapeDtypeStruct((1, _F), jnp.float32),
        ],
    )(x, parts)


def _bn_from_stats(u, s, q, gamma, beta):
    m = s / _N
    v = q / _N - m * m
    return (u - m) / jnp.sqrt(v + _EPS) * gamma + beta


def _mid_body(u_ref, s_ref, q_ref, g_ref, b_ref, wd_ref, ws_ref,
              h_ref, td_ref, ts_ref):
    h = _bn_from_stats(u_ref[...], s_ref[...], q_ref[...],
                       g_ref[...], b_ref[...])
    h_ref[...] = h
    td_ref[...] = _dot(h, wd_ref[...])
    ts_ref[...] = _dot(h, ws_ref[...])


def _mid(u, s, q, gamma, beta, Wd, Wsrc):
    return pl.pallas_call(
        _mid_body,
        grid=(_NRB,),
        in_specs=[
            pl.BlockSpec((_RB, _F), lambda i: (i, 0)),
            pl.BlockSpec((1, _F), lambda i: (0, 0)),
            pl.BlockSpec((1, _F), lambda i: (0, 0)),
            pl.BlockSpec((1, _F), lambda i: (0, 0)),
            pl.BlockSpec((1, _F), lambda i: (0, 0)),
            pl.BlockSpec((_F, 2 * _F), lambda i: (0, 0)),
            pl.BlockSpec((_F, 2 * _F), lambda i: (0, 0)),
        ],
        out_specs=[
            pl.BlockSpec((_RB, _F), lambda i: (i, 0)),
            pl.BlockSpec((_RB, 2 * _F), lambda i: (i, 0)),
            pl.BlockSpec((_RB, 2 * _F), lambda i: (i, 0)),
        ],
        out_shape=[
            jax.ShapeDtypeStruct((_N, _F), jnp.float32),
            jax.ShapeDtypeStruct((_N, 2 * _F), jnp.float32),
            jax.ShapeDtypeStruct((_N, 2 * _F), jnp.float32),
        ],
    )(u, s, q, gamma.reshape(1, -1), beta.reshape(1, -1), Wd, Wsrc)


def _final_body(u_ref, s_ref, q_ref, g_ref, b_ref, seg_ref, wfc_ref, bfc_ref,
                o_ref, sacc_ref, cacc_ref):
    h2 = _bn_from_stats(u_ref[...], s_ref[...], q_ref[...],
                        g_ref[...], b_ref[...])
    onehot = (seg_ref[...] == lax.broadcasted_iota(jnp.int32, (_RB, _G), 1)
              ).astype(jnp.float32)
    ps = lax.dot_general(onehot, h2, (((0,), (0,)), ((), ())),
                         precision=_PREC, preferred_element_type=jnp.float32)
    pc = jnp.sum(onehot, axis=0)[:, None]
    i = pl.program_id(0)

    @pl.when(i == 0)
    def _init():
        sacc_ref[...] = ps
        cacc_ref[...] = pc

    @pl.when(i > 0)
    def _acc():
        sacc_ref[...] += ps
        cacc_ref[...] += pc

    @pl.when(i == _NRB - 1)
    def _emit():
        pooled = sacc_ref[...] / jnp.clip(cacc_ref[...], 1.0, None)
        o_ref[...] = _dot(pooled, wfc_ref[...]) + bfc_ref[...]


def _final(u, s, q, gamma, beta, seg, Wfc, bfc):
    return pl.pallas_call(
        _final_body,
        grid=(_NRB,),
        in_specs=[
            pl.BlockSpec((_RB, _F), lambda i: (i, 0)),
            pl.BlockSpec((1, _F), lambda i: (0, 0)),
            pl.BlockSpec((1, _F), lambda i: (0, 0)),
            pl.BlockSpec((1, _F), lambda i: (0, 0)),
            pl.BlockSpec((1, _F), lambda i: (0, 0)),
            pl.BlockSpec((_RB, 1), lambda i: (i, 0)),
            pl.BlockSpec((_F, _C), lambda i: (0, 0)),
            pl.BlockSpec((1, _C), lambda i: (0, 0)),
        ],
        out_specs=pl.BlockSpec((_G, _C), lambda i: (0, 0)),
        out_shape=jax.ShapeDtypeStruct((_G, _C), jnp.float32),
        scratch_shapes=[
            pltpu.VMEM((_G, _F), jnp.float32),
            pltpu.VMEM((_G, 1), jnp.float32),
        ],
    )(u, s, q, gamma.reshape(1, -1), beta.reshape(1, -1),
      seg.reshape(_N, 1), Wfc, bfc.reshape(1, -1))


def _softplus(x):
    u = jnp.exp(-jnp.abs(x))
    q = jnp.float32(_Q[-1])
    for c in _Q[-2::-1]:
        q = q * u + jnp.float32(c)
    return jnp.maximum(x, 0.0) + q * u


def _sc_edge_body(td_hbm, ts_hbm, ea_hbm, src_hbm, dst_hbm, out_hbm,
                  dsti, srci, gd, gs, eav, msg, zbuf, agg, sem1, sem2):
    c = lax.axis_index("c")
    s = lax.axis_index("s")
    wid = c * _NS + s

    @pl.loop(0, _ZROWS)
    def _zero(i):
        for j in range(_F // 16):
            zbuf[i, pl.ds(j * 16, 16)] = jnp.zeros((16,), jnp.float32)

    for t in range(_ROWS_W // _ZROWS):
        pltpu.sync_copy(zbuf, agg.at[pl.ds(s * _ROWS_W + t * _ZROWS, _ZROWS)])

    @pl.when(s == 0)
    def _zero_tail():
        pltpu.sync_copy(zbuf.at[pl.ds(0, _TAIL)],
                        agg.at[pl.ds(_NS * _ROWS_W, _TAIL)])

    plsc.subcore_barrier()

    base = wid * _PER_W

    @pl.loop(0, _CHUNKS)
    def _chunk(k):
        off = base + k * _B
        pltpu.sync_copy(dst_hbm.at[pl.ds(off, _B)], dsti)
        pltpu.sync_copy(src_hbm.at[pl.ds(off, _B)], srci)
        cp1 = pltpu.async_copy(td_hbm.at[dsti], gd, sem1)
        cp2 = pltpu.async_copy(ts_hbm.at[srci], gs, sem2)
        pltpu.sync_copy(ea_hbm.at[pl.ds(off, _B)], eav)
        cp1.wait()
        cp2.wait()

        @pl.loop(0, _B)
        def _edge(i):
            for j in range(_F // 16):
                slf = pl.ds(j * 16, 16)
                sls = pl.ds(_F + j * 16, 16)
                zf = gd[i, slf] + gs[i, slf] + eav[i, slf]
                zs = gd[i, sls] + gs[i, sls] + eav[i, sls]
                gate = 1.0 / (1.0 + jnp.exp(-zf))
                msg[i, slf] = gate * _softplus(zs)

        pltpu.sync_copy(msg, agg.at[dsti], add=True)

    plsc.subcore_barrier()
    pltpu.sync_copy(agg.at[pl.ds(s * _ROWS_W, _ROWS_W)],
                    out_hbm.at[c, pl.ds(s * _ROWS_W, _ROWS_W)])

    @pl.when(s == 0)
    def _write_tail():
        pltpu.sync_copy(agg.at[pl.ds(_NS * _ROWS_W, _TAIL)],
                        out_hbm.at[c, pl.ds(_NS * _ROWS_W, _TAIL)])


def _sc_edge(Tdst, Tsrc, EA, src, dst):
    mesh = plsc.VectorSubcoreMesh(
        core_axis_name="c", subcore_axis_name="s",
        num_cores=_NC, num_subcores=_NS)
    fn = pl.kernel(
        _sc_edge_body,
        out_type=jax.ShapeDtypeStruct((_NC, _N, _F), jnp.float32),
        mesh=mesh,
        scratch_types=[
            pltpu.VMEM((_B,), jnp.int32),
            pltpu.VMEM((_B,), jnp.int32),
            pltpu.VMEM((_B, 2 * _F), jnp.float32),
            pltpu.VMEM((_B, 2 * _F), jnp.float32),
            pltpu.VMEM((_B, 2 * _F), jnp.float32),
            pltpu.VMEM((_B, _F), jnp.float32),
            pltpu.VMEM((_ZROWS, _F), jnp.float32),
            pltpu.VMEM_SHARED((_N, _F), jnp.float32),
            pltpu.SemaphoreType.DMA,
            pltpu.SemaphoreType.DMA,
        ],
    )
    return fn(Tdst, Tsrc, EA, src, dst)


def kernel(x, edge_index, edge_attr, batch, Wf0, bf0, Ws0, bs0,
           Wf1, bf1, Ws1, bs1, gamma0, beta0, gamma1, beta1, Wfc, bfc):
    src = edge_index[0].astype(jnp.int32)
    dst = edge_index[1].astype(jnp.int32)
    seg = batch.astype(jnp.int32)

    def split(Wf, Ws):
        Wd = jnp.concatenate([Wf[:_F], Ws[:_F]], axis=1)
        Wsrc = jnp.concatenate([Wf[_F:2 * _F], Ws[_F:2 * _F]], axis=1)
        We = jnp.concatenate([Wf[2 * _F:], Ws[2 * _F:]], axis=1)
        return Wd, Wsrc, We

    Wd0, Wsrc0, We0 = split(Wf0, Ws0)
    Wd1, Wsrc1, We1 = split(Wf1, Ws1)
    b0 = jnp.concatenate([bf0, bs0])
    b1 = jnp.concatenate([bf1, bs1])

    EA0, EA1 = _edge_terms(edge_attr, We0, b0, We1, b1)

    Td0, Ts0 = _tables(x, Wd0, Wsrc0)
    parts0 = _sc_edge(Td0, Ts0, EA0, src, dst)

    u0, s0, q0 = _stats(x, parts0)
    h1, Td1, Ts1 = _mid(u0, s0, q0, gamma0, beta0, Wd1, Wsrc1)
    parts1 = _sc_edge(Td1, Ts1, EA1, src, dst)

    u1, s1, q1 = _stats(h1, parts1)
    return _final(u1, s1, q1, gamma1, beta1, seg, Wfc, bfc)


# speedup vs baseline: 3.5084x; 3.5084x over previous
import dataclasses
import functools

import jax
import jax.numpy as jnp
from jax import lax
from jax.experimental import pallas as pl
from jax.experimental.pallas import tpu as pltpu
from jax.experimental.pallas import tpu_sc as plsc

_N = 10000
_E = 320000
_F = 128
_D = 16
_G = 64
_C = 16
_EPS = 1e-5

_NC = 2
_NS = 16
_NW = _NC * _NS
_PER_W = _E // _NW
_B = 40
_CHUNKS = _PER_W // _B
_ROWS_W = 624
_TAIL = _N - _NS * _ROWS_W
_ZROWS = 48

_Q = (0.9999999992732383, -0.49999981736734733, 0.33332569426223596,
      -0.24987394966920914, 0.19891548439724996, -0.16108664201878709,
      0.12425161741270922, -0.08253599813002745, 0.04155807546752951,
      -0.013444998519934246, 0.0020377159799453265)

_PREC = lax.Precision.HIGHEST


def _dot(a, b):
    return jnp.dot(a, b, preferred_element_type=jnp.float32, precision=_PREC)


def _pack_words(a):
    pieces = []
    for g in range(2 * _F // 32):
        lo = a[:, 32 * g:32 * g + 16]
        hi = a[:, 32 * g + 16:32 * g + 32]
        lob = lax.bitcast_convert_type(
            lo.astype(jnp.bfloat16).astype(jnp.float32), jnp.int32)
        hib = lax.bitcast_convert_type(
            hi.astype(jnp.bfloat16).astype(jnp.float32), jnp.int32)
        pieces.append(jnp.bitwise_or(jnp.bitwise_and(hib, jnp.int32(-65536)),
                                     lax.shift_right_logical(lob, 16)))
    return jnp.concatenate(pieces, axis=1)


def _edge_terms_body(ea_ref, w0_ref, b0_ref, w1_ref, b1_ref, o0_ref, o1_ref):
    ea = ea_ref[...]
    o0_ref[...] = _pack_words(_dot(ea, w0_ref[...]) + b0_ref[...])
    o1_ref[...] = _pack_words(_dot(ea, w1_ref[...]) + b1_ref[...])


def _edge_terms(edge_attr, We0, b0, We1, b1):
    eb = 4000
    return pl.pallas_call(
        _edge_terms_body,
        grid=(_E // eb,),
        in_specs=[
            pl.BlockSpec((eb, _D), lambda i: (i, 0)),
            pl.BlockSpec((_D, 2 * _F), lambda i: (0, 0)),
            pl.BlockSpec((1, 2 * _F), lambda i: (0, 0)),
            pl.BlockSpec((_D, 2 * _F), lambda i: (0, 0)),
            pl.BlockSpec((1, 2 * _F), lambda i: (0, 0)),
        ],
        out_specs=[
            pl.BlockSpec((eb, _F), lambda i: (i, 0)),
            pl.BlockSpec((eb, _F), lambda i: (i, 0)),
        ],
        out_shape=[jax.ShapeDtypeStruct((_E, _F), jnp.int32)] * 2,
    )(edge_attr, We0, b0.reshape(1, -1), We1, b1.reshape(1, -1))


_TRB = 2000


def _tables_body(x_ref, w2_ref, t2_ref):
    t2_ref[...] = _pack_words(_dot(x_ref[...], w2_ref[0]))


def _tables(h, W2):
    nrb = _N // _TRB
    return pl.pallas_call(
        _tables_body,
        grid=(2, nrb),
        in_specs=[
            pl.BlockSpec((_TRB, _F), lambda t, i: (i, 0)),
            pl.BlockSpec((1, _F, 2 * _F), lambda t, i: (t, 0, 0)),
        ],
        out_specs=pl.BlockSpec((_TRB, _F), lambda t, i: (t * nrb + i, 0)),
        out_shape=jax.ShapeDtypeStruct((2 * _N, _F), jnp.int32),
    )(h, W2)


_RB = 2000
_NRB = _N // _RB


def _stats_body(x_ref, p_ref, u_ref, sum_ref, sq_ref):
    u = x_ref[...] + p_ref[0] + p_ref[1]
    u_ref[...] = u
    ps = jnp.sum(u, axis=0, keepdims=True)
    pq = jnp.sum(u * u, axis=0, keepdims=True)

    @pl.when(pl.program_id(0) == 0)
    def _init():
        sum_ref[...] = ps
        sq_ref[...] = pq

    @pl.when(pl.program_id(0) > 0)
    def _acc():
        sum_ref[...] += ps
        sq_ref[...] += pq


def _stats(x, parts):
    return pl.pallas_call(
        _stats_body,
        grid=(_NRB,),
        in_specs=[
            pl.BlockSpec((_RB, _F), lambda i: (i, 0)),
            pl.BlockSpec((2, _RB, _F), lambda i: (0, i, 0)),
        ],
        out_specs=[
            pl.BlockSpec((_RB, _F), lambda i: (i, 0)),
            pl.BlockSpec((1, _F), lambda i: (0, 0)),
            pl.BlockSpec((1, _F), lambda i: (0, 0)),
        ],
        out_shape=[
            jax.ShapeDtypeStruct((_N, _F), jnp.float32),
            jax.ShapeDtypeStruct((1, _F), jnp.float32),
            jax.ShapeDtypeStruct((1, _F), jnp.float32),
        ],
    )(x, parts)


def _bn_from_stats(u, s, q, gamma, beta):
    m = s / _N
    v = q / _N - m * m
    return (u - m) / jnp.sqrt(v + _EPS) * gamma + beta


def _mid_body(u_ref, s_ref, q_ref, g_ref, b_ref, w2_ref, h_ref, t2_ref):
    h = _bn_from_stats(u_ref[...], s_ref[...], q_ref[...],
                       g_ref[...], b_ref[...])
    h_ref[...] = h
    t2_ref[...] = _pack_words(_dot(h, w2_ref[0]))


def _mid(u, s, q, gamma, beta, W2):
    nrb = _N // _RB
    return pl.pallas_call(
        _mid_body,
        grid=(2, nrb),
        in_specs=[
            pl.BlockSpec((_RB, _F), lambda t, i: (i, 0)),
            pl.BlockSpec((1, _F), lambda t, i: (0, 0)),
            pl.BlockSpec((1, _F), lambda t, i: (0, 0)),
            pl.BlockSpec((1, _F), lambda t, i: (0, 0)),
            pl.BlockSpec((1, _F), lambda t, i: (0, 0)),
            pl.BlockSpec((1, _F, 2 * _F), lambda t, i: (t, 0, 0)),
        ],
        out_specs=[
            pl.BlockSpec((_RB, _F), lambda t, i: (i, 0)),
            pl.BlockSpec((_RB, _F), lambda t, i: (t * nrb + i, 0)),
        ],
        out_shape=[
            jax.ShapeDtypeStruct((_N, _F), jnp.float32),
            jax.ShapeDtypeStruct((2 * _N, _F), jnp.int32),
        ],
    )(u, s, q, gamma.reshape(1, -1), beta.reshape(1, -1), W2)


def _final_body(u_ref, s_ref, q_ref, g_ref, b_ref, seg_ref, wfc_ref, bfc_ref,
                o_ref, sacc_ref, cacc_ref):
    h2 = _bn_from_stats(u_ref[...], s_ref[...], q_ref[...],
                        g_ref[...], b_ref[...])
    onehot = (seg_ref[...] == lax.broadcasted_iota(jnp.int32, (_RB, _G), 1)
              ).astype(jnp.float32)
    ps = lax.dot_general(onehot, h2, (((0,), (0,)), ((), ())),
                         precision=_PREC, preferred_element_type=jnp.float32)
    pc = jnp.sum(onehot, axis=0)[:, None]
    i = pl.program_id(0)

    @pl.when(i == 0)
    def _init():
        sacc_ref[...] = ps
        cacc_ref[...] = pc

    @pl.when(i > 0)
    def _acc():
        sacc_ref[...] += ps
        cacc_ref[...] += pc

    @pl.when(i == _NRB - 1)
    def _emit():
        pooled = sacc_ref[...] / jnp.clip(cacc_ref[...], 1.0, None)
        o_ref[...] = _dot(pooled, wfc_ref[...]) + bfc_ref[...]


def _final(u, s, q, gamma, beta, seg, Wfc, bfc):
    return pl.pallas_call(
        _final_body,
        grid=(_NRB,),
        in_specs=[
            pl.BlockSpec((_RB, _F), lambda i: (i, 0)),
            pl.BlockSpec((1, _F), lambda i: (0, 0)),
            pl.BlockSpec((1, _F), lambda i: (0, 0)),
            pl.BlockSpec((1, _F), lambda i: (0, 0)),
            pl.BlockSpec((1, _F), lambda i: (0, 0)),
            pl.BlockSpec((_RB, 1), lambda i: (i, 0)),
            pl.BlockSpec((_F, _C), lambda i: (0, 0)),
            pl.BlockSpec((1, _C), lambda i: (0, 0)),
        ],
        out_specs=pl.BlockSpec((_G, _C), lambda i: (0, 0)),
        out_shape=jax.ShapeDtypeStruct((_G, _C), jnp.float32),
        scratch_shapes=[
            pltpu.VMEM((_G, _F), jnp.float32),
            pltpu.VMEM((_G, 1), jnp.float32),
        ],
    )(u, s, q, gamma.reshape(1, -1), beta.reshape(1, -1),
      seg.reshape(_N, 1), Wfc, bfc.reshape(1, -1))


def _softplus(x):
    u = jnp.exp(-jnp.abs(x))
    q = jnp.float32(_Q[-1])
    for c in _Q[-2::-1]:
        q = q * u + jnp.float32(c)
    return jnp.maximum(x, 0.0) + q * u


def _lo_f32(w):
    return plsc.bitcast(jnp.left_shift(w, 16), jnp.float32)


def _hi_f32(w):
    return plsc.bitcast(jnp.bitwise_and(w, jnp.int32(-65536)), jnp.float32)


def _w16(ref, i, wcol):
    return ref[i, pl.ds(wcol, 16)]


def _sc_edge_body(t2_hbm, ea_hbm, dst_hbm, srcp_hbm, out_hbm,
                  idx, dsts, g2, eav, msg, zbuf, agg, semi, semg, seme, sems):
    c = lax.axis_index("c")
    s = lax.axis_index("s")
    base = (c * _NS + s) * _PER_W

    def fire_idx(k, r):
        off = base + k * _B
        pltpu.async_copy(dst_hbm.at[pl.ds(off, _B)],
                         idx[r].at[pl.ds(0, _B)], semi[r])
        pltpu.async_copy(srcp_hbm.at[pl.ds(off, _B)],
                         idx[r].at[pl.ds(_B, _B)], semi[r])

    def wait_idx(k, r):
        off = base + k * _B
        pltpu.make_async_copy(dst_hbm.at[pl.ds(off, _B)],
                              idx[r].at[pl.ds(0, _B)], semi[r]).wait()
        pltpu.make_async_copy(srcp_hbm.at[pl.ds(off, _B)],
                              idx[r].at[pl.ds(_B, _B)], semi[r]).wait()

    def fire_gather(k, r):
        off = base + k * _B
        pltpu.async_copy(t2_hbm.at[idx[r]], g2[r], semg[r])
        pltpu.async_copy(ea_hbm.at[pl.ds(off, _B)], eav[r], seme[r])

    def wait_gather(k, r):
        off = base + k * _B
        pltpu.make_async_copy(t2_hbm.at[idx[r]], g2[r], semg[r]).wait()
        pltpu.make_async_copy(ea_hbm.at[pl.ds(off, _B)], eav[r],
                              seme[r]).wait()

    def fire_scatter(r):
        pltpu.async_copy(msg[r], agg.at[dsts[r]], sems[r], add=True)

    def wait_scatter(r):
        pltpu.make_async_copy(msg[r], agg.at[dsts[r]], sems[r]).wait()

    def copy_dsts(r):
        dsts[r][pl.ds(0, 16)] = idx[r][pl.ds(0, 16)]
        dsts[r][pl.ds(16, 16)] = idx[r][pl.ds(16, 16)]
        dsts[r][pl.ds(24, 16)] = idx[r][pl.ds(24, 16)]

    def compute(r):
        @pl.loop(0, _B)
        def _edge(i):
            for g in range(_F // 32):
                cf = 32 * g
                wf = 16 * g
                ws = _F // 2 + 16 * g
                wdf = _w16(g2[r], i, wf)
                wsf = _w16(g2[r], _B + i, wf)
                wef = _w16(eav[r], i, wf)
                wds = _w16(g2[r], i, ws)
                wss = _w16(g2[r], _B + i, ws)
                wes = _w16(eav[r], i, ws)
                for h, ext in enumerate((_lo_f32, _hi_f32)):
                    zf = ext(wdf) + ext(wsf) + ext(wef)
                    zs = ext(wds) + ext(wss) + ext(wes)
                    gate = 1.0 / (1.0 + jnp.exp(-zf))
                    msg[r][i, pl.ds(cf + 16 * h, 16)] = gate * _softplus(zs)

    fire_idx(0, 0)
    fire_idx(1, 1)

    @pl.loop(0, _ZROWS)
    def _zero(i):
        for j in range(_F // 16):
            zbuf[i, pl.ds(j * 16, 16)] = jnp.zeros((16,), jnp.float32)

    for t in range(_ROWS_W // _ZROWS):
        pltpu.sync_copy(zbuf, agg.at[pl.ds(s * _ROWS_W + t * _ZROWS, _ZROWS)])

    @pl.when(s == 0)
    def _zero_tail():
        pltpu.sync_copy(zbuf.at[pl.ds(0, _TAIL)],
                        agg.at[pl.ds(_NS * _ROWS_W, _TAIL)])

    wait_idx(0, 0)
    fire_gather(0, 0)
    plsc.subcore_barrier()

    wait_idx(1, 1)
    fire_gather(1, 1)
    wait_gather(0, 0)
    copy_dsts(0)
    fire_idx(2, 0)
    compute(0)
    fire_scatter(0)

    wait_idx(2, 0)
    fire_gather(2, 0)
    wait_gather(1, 1)
    copy_dsts(1)
    fire_idx(3, 1)
    compute(1)
    fire_scatter(1)

    @pl.loop(0, (_CHUNKS - 4) // 2)
    def _pair(m):
        k = 2 * m + 2
        for p in range(2):
            r = p
            rn = 1 - p
            wait_idx(k + p + 1, rn)
            fire_gather(k + p + 1, rn)
            wait_gather(k + p, r)
            wait_scatter(r)
            copy_dsts(r)
            fire_idx(k + p + 2, r)
            compute(r)
            fire_scatter(r)

    wait_idx(_CHUNKS - 1, 1)
    fire_gather(_CHUNKS - 1, 1)
    wait_gather(_CHUNKS - 2, 0)
    wait_scatter(0)
    copy_dsts(0)
    compute(0)
    fire_scatter(0)

    wait_gather(_CHUNKS - 1, 1)
    wait_scatter(1)
    copy_dsts(1)
    compute(1)
    fire_scatter(1)

    wait_scatter(0)
    wait_scatter(1)
    plsc.subcore_barrier()

    pltpu.sync_copy(agg.at[pl.ds(s * _ROWS_W, _ROWS_W)],
                    out_hbm.at[c, pl.ds(s * _ROWS_W, _ROWS_W)])

    @pl.when(s == 0)
    def _write_tail():
        pltpu.sync_copy(agg.at[pl.ds(_NS * _ROWS_W, _TAIL)],
                        out_hbm.at[c, pl.ds(_NS * _ROWS_W, _TAIL)])


def _sc_edge(T2, EA, dst, srcp):
    mesh = plsc.VectorSubcoreMesh(
        core_axis_name="c", subcore_axis_name="s",
        num_cores=_NC, num_subcores=_NS)
    cp = pltpu.CompilerParams()
    if "needs_layout_passes" in pltpu.CompilerParams.__dataclass_fields__:
        cp = dataclasses.replace(cp, needs_layout_passes=False)
    fn = pl.kernel(
        _sc_edge_body,
        out_type=jax.ShapeDtypeStruct((_NC, _N, _F), jnp.float32),
        mesh=mesh,
        compiler_params=cp,
        scratch_types=[
            [pltpu.VMEM((2 * _B,), jnp.int32)] * 2,
            [pltpu.VMEM((_B,), jnp.int32)] * 2,
            [pltpu.VMEM((2 * _B, _F), jnp.int32)] * 2,
            [pltpu.VMEM((_B, _F), jnp.int32)] * 2,
            [pltpu.VMEM((_B, _F), jnp.float32)] * 2,
            pltpu.VMEM((_ZROWS, _F), jnp.float32),
            pltpu.VMEM_SHARED((_N, _F), jnp.float32),
            [pltpu.SemaphoreType.DMA] * 2,
            [pltpu.SemaphoreType.DMA] * 2,
            [pltpu.SemaphoreType.DMA] * 2,
            [pltpu.SemaphoreType.DMA] * 2,
        ],
    )
    return fn(T2, EA, dst, srcp)


def kernel(x, edge_index, edge_attr, batch, Wf0, bf0, Ws0, bs0,
           Wf1, bf1, Ws1, bs1, gamma0, beta0, gamma1, beta1, Wfc, bfc):
    src = edge_index[0].astype(jnp.int32)
    dst = edge_index[1].astype(jnp.int32)
    srcp = src + _N
    seg = batch.astype(jnp.int32)

    def split(Wf, Ws):
        Wd = jnp.concatenate([Wf[:_F], Ws[:_F]], axis=1)
        Wsrc = jnp.concatenate([Wf[_F:2 * _F], Ws[_F:2 * _F]], axis=1)
        We = jnp.concatenate([Wf[2 * _F:], Ws[2 * _F:]], axis=1)
        return Wd, Wsrc, We

    Wd0, Wsrc0, We0 = split(Wf0, Ws0)
    Wd1, Wsrc1, We1 = split(Wf1, Ws1)
    b0 = jnp.concatenate([bf0, bs0])
    b1 = jnp.concatenate([bf1, bs1])

    EA0, EA1 = _edge_terms(edge_attr, We0, b0, We1, b1)

    T20 = _tables(x, jnp.stack([Wd0, Wsrc0]))
    parts0 = _sc_edge(T20, EA0, dst, srcp)

    u0, s0, q0 = _stats(x, parts0)
    h1, T21 = _mid(u0, s0, q0, gamma0, beta0, jnp.stack([Wd1, Wsrc1]))
    parts1 = _sc_edge(T21, EA1, dst, srcp)

    u1, s1, q1 = _stats(h1, parts1)
    return _final(u1, s1, q1, gamma1, beta1, seg, Wfc, bfc)


# speedup vs baseline: 5.3458x; 1.5237x over previous
import dataclasses
import functools

import jax
import jax.numpy as jnp
from jax import lax
from jax.experimental import pallas as pl
from jax.experimental.pallas import tpu as pltpu
from jax.experimental.pallas import tpu_sc as plsc

_N = 10000
_E = 320000
_F = 128
_D = 16
_G = 64
_C = 16
_EPS = 1e-5

_NC = 2
_NS = 16
_NW = _NC * _NS
_PER_W = _E // _NW
_B = 40
_CHUNKS = _PER_W // _B
_ROWS_W = 624
_TAIL = _N - _NS * _ROWS_W
_ZROWS = 48

_Q = (0.9999999992732383, -0.49999981736734733, 0.33332569426223596,
      -0.24987394966920914, 0.19891548439724996, -0.16108664201878709,
      0.12425161741270922, -0.08253599813002745, 0.04155807546752951,
      -0.013444998519934246, 0.0020377159799453265)

_PREC = lax.Precision.HIGHEST


def _dot(a, b):
    return jnp.dot(a, b, preferred_element_type=jnp.float32, precision=_PREC)


_LOPERM = tuple(32 * g + t for g in range(8) for t in range(16))
_HIPERM = tuple(32 * g + 16 + t for g in range(8) for t in range(16))


def _pack2(lo, hi):
    lob = lax.bitcast_convert_type(
        lo.astype(jnp.bfloat16).astype(jnp.float32), jnp.int32)
    hib = lax.bitcast_convert_type(
        hi.astype(jnp.bfloat16).astype(jnp.float32), jnp.int32)
    return jnp.bitwise_or(jnp.bitwise_and(hib, jnp.int32(-65536)),
                          lax.shift_right_logical(lob, 16))


def _edge_terms_body(ea_ref, wlo_ref, blo_ref, whi_ref, bhi_ref, o_ref):
    ea = ea_ref[...]
    o_ref[...] = _pack2(_dot(ea, wlo_ref[...]) + blo_ref[...],
                        _dot(ea, whi_ref[...]) + bhi_ref[...])


def _edge_terms(edge_attr, Wlo, blo, Whi, bhi):
    eb = 4000
    return pl.pallas_call(
        _edge_terms_body,
        grid=(_E // eb,),
        in_specs=[
            pl.BlockSpec((eb, _D), lambda i: (i, 0)),
            pl.BlockSpec((_D, _F), lambda i: (0, 0)),
            pl.BlockSpec((1, _F), lambda i: (0, 0)),
            pl.BlockSpec((_D, _F), lambda i: (0, 0)),
            pl.BlockSpec((1, _F), lambda i: (0, 0)),
        ],
        out_specs=pl.BlockSpec((eb, _F), lambda i: (i, 0)),
        out_shape=jax.ShapeDtypeStruct((_E, _F), jnp.int32),
    )(edge_attr, Wlo, blo.reshape(1, -1), Whi, bhi.reshape(1, -1))


_TRB = 2000


def _tables_body(x_ref, wlo_ref, whi_ref, t2_ref):
    xv = x_ref[...]
    t2_ref[...] = _pack2(_dot(xv, wlo_ref[0]), _dot(xv, whi_ref[0]))


def _tables(h, W2lo, W2hi):
    nrb = _N // _TRB
    return pl.pallas_call(
        _tables_body,
        grid=(2, nrb),
        in_specs=[
            pl.BlockSpec((_TRB, _F), lambda t, i: (i, 0)),
            pl.BlockSpec((1, _F, _F), lambda t, i: (t, 0, 0)),
            pl.BlockSpec((1, _F, _F), lambda t, i: (t, 0, 0)),
        ],
        out_specs=pl.BlockSpec((_TRB, _F), lambda t, i: (t * nrb + i, 0)),
        out_shape=jax.ShapeDtypeStruct((2 * _N, _F), jnp.int32),
    )(h, W2lo, W2hi)


_RB = 2000
_NRB = _N // _RB


def _stats_body(x_ref, p_ref, u_ref, sum_ref, sq_ref):
    u = x_ref[...] + p_ref[0] + p_ref[1]
    u_ref[...] = u
    ps = jnp.sum(u, axis=0, keepdims=True)
    pq = jnp.sum(u * u, axis=0, keepdims=True)

    @pl.when(pl.program_id(0) == 0)
    def _init():
        sum_ref[...] = ps
        sq_ref[...] = pq

    @pl.when(pl.program_id(0) > 0)
    def _acc():
        sum_ref[...] += ps
        sq_ref[...] += pq


def _stats(x, parts):
    return pl.pallas_call(
        _stats_body,
        grid=(_NRB,),
        in_specs=[
            pl.BlockSpec((_RB, _F), lambda i: (i, 0)),
            pl.BlockSpec((2, _RB, _F), lambda i: (0, i, 0)),
        ],
        out_specs=[
            pl.BlockSpec((_RB, _F), lambda i: (i, 0)),
            pl.BlockSpec((1, _F), lambda i: (0, 0)),
            pl.BlockSpec((1, _F), lambda i: (0, 0)),
        ],
        out_shape=[
            jax.ShapeDtypeStruct((_N, _F), jnp.float32),
            jax.ShapeDtypeStruct((1, _F), jnp.float32),
            jax.ShapeDtypeStruct((1, _F), jnp.float32),
        ],
    )(x, parts)


def _bn_from_stats(u, s, q, gamma, beta):
    m = s / _N
    v = q / _N - m * m
    return (u - m) / jnp.sqrt(v + _EPS) * gamma + beta


def _mid_body(u_ref, s_ref, q_ref, g_ref, b_ref, wlo_ref, whi_ref,
              h_ref, t2_ref):
    h = _bn_from_stats(u_ref[...], s_ref[...], q_ref[...],
                       g_ref[...], b_ref[...])
    h_ref[...] = h
    t2_ref[...] = _pack2(_dot(h, wlo_ref[0]), _dot(h, whi_ref[0]))


def _mid(u, s, q, gamma, beta, W2lo, W2hi):
    nrb = _N // _RB
    return pl.pallas_call(
        _mid_body,
        grid=(2, nrb),
        in_specs=[
            pl.BlockSpec((_RB, _F), lambda t, i: (i, 0)),
            pl.BlockSpec((1, _F), lambda t, i: (0, 0)),
            pl.BlockSpec((1, _F), lambda t, i: (0, 0)),
            pl.BlockSpec((1, _F), lambda t, i: (0, 0)),
            pl.BlockSpec((1, _F), lambda t, i: (0, 0)),
            pl.BlockSpec((1, _F, _F), lambda t, i: (t, 0, 0)),
            pl.BlockSpec((1, _F, _F), lambda t, i: (t, 0, 0)),
        ],
        out_specs=[
            pl.BlockSpec((_RB, _F), lambda t, i: (i, 0)),
            pl.BlockSpec((_RB, _F), lambda t, i: (t * nrb + i, 0)),
        ],
        out_shape=[
            jax.ShapeDtypeStruct((_N, _F), jnp.float32),
            jax.ShapeDtypeStruct((2 * _N, _F), jnp.int32),
        ],
    )(u, s, q, gamma.reshape(1, -1), beta.reshape(1, -1), W2lo, W2hi)


def _final_body(u_ref, s_ref, q_ref, g_ref, b_ref, seg_ref, wfc_ref, bfc_ref,
                o_ref, sacc_ref, cacc_ref):
    h2 = _bn_from_stats(u_ref[...], s_ref[...], q_ref[...],
                        g_ref[...], b_ref[...])
    onehot = (seg_ref[...] == lax.broadcasted_iota(jnp.int32, (_RB, _G), 1)
              ).astype(jnp.float32)
    ps = lax.dot_general(onehot, h2, (((0,), (0,)), ((), ())),
                         precision=_PREC, preferred_element_type=jnp.float32)
    pc = jnp.sum(onehot, axis=0)[:, None]
    i = pl.program_id(0)

    @pl.when(i == 0)
    def _init():
        sacc_ref[...] = ps
        cacc_ref[...] = pc

    @pl.when(i > 0)
    def _acc():
        sacc_ref[...] += ps
        cacc_ref[...] += pc

    @pl.when(i == _NRB - 1)
    def _emit():
        pooled = sacc_ref[...] / jnp.clip(cacc_ref[...], 1.0, None)
        o_ref[...] = _dot(pooled, wfc_ref[...]) + bfc_ref[...]


def _final(u, s, q, gamma, beta, seg, Wfc, bfc):
    return pl.pallas_call(
        _final_body,
        grid=(_NRB,),
        in_specs=[
            pl.BlockSpec((_RB, _F), lambda i: (i, 0)),
            pl.BlockSpec((1, _F), lambda i: (0, 0)),
            pl.BlockSpec((1, _F), lambda i: (0, 0)),
            pl.BlockSpec((1, _F), lambda i: (0, 0)),
            pl.BlockSpec((1, _F), lambda i: (0, 0)),
            pl.BlockSpec((_RB, 1), lambda i: (i, 0)),
            pl.BlockSpec((_F, _C), lambda i: (0, 0)),
            pl.BlockSpec((1, _C), lambda i: (0, 0)),
        ],
        out_specs=pl.BlockSpec((_G, _C), lambda i: (0, 0)),
        out_shape=jax.ShapeDtypeStruct((_G, _C), jnp.float32),
        scratch_shapes=[
            pltpu.VMEM((_G, _F), jnp.float32),
            pltpu.VMEM((_G, 1), jnp.float32),
        ],
    )(u, s, q, gamma.reshape(1, -1), beta.reshape(1, -1),
      seg.reshape(_N, 1), Wfc, bfc.reshape(1, -1))


def _softplus(x):
    u = jnp.exp(-jnp.abs(x))
    q = jnp.float32(_Q[-1])
    for c in _Q[-2::-1]:
        q = q * u + jnp.float32(c)
    return jnp.maximum(x, 0.0) + q * u


def _lo_f32(w):
    return plsc.bitcast(jnp.left_shift(w, 16), jnp.float32)


def _hi_f32(w):
    return plsc.bitcast(jnp.bitwise_and(w, jnp.int32(-65536)), jnp.float32)


def _w16(ref, i, wcol):
    return ref[i, pl.ds(wcol, 16)]


def _sc_edge_body(t2_hbm, ea_hbm, dst_hbm, srcp_hbm, out_hbm,
                  idx, dsts, g2, eav, msg, zbuf, agg, semi, semg, seme, sems):
    c = lax.axis_index("c")
    s = lax.axis_index("s")
    base = (c * _NS + s) * _PER_W

    def fire_idx(k, r):
        off = base + k * _B
        pltpu.async_copy(dst_hbm.at[pl.ds(off, _B)],
                         idx[r].at[pl.ds(0, _B)], semi[r])
        pltpu.async_copy(srcp_hbm.at[pl.ds(off, _B)],
                         idx[r].at[pl.ds(_B, _B)], semi[r])

    def wait_idx(k, r):
        off = base + k * _B
        pltpu.make_async_copy(dst_hbm.at[pl.ds(off, _B)],
                              idx[r].at[pl.ds(0, _B)], semi[r]).wait()
        pltpu.make_async_copy(srcp_hbm.at[pl.ds(off, _B)],
                              idx[r].at[pl.ds(_B, _B)], semi[r]).wait()

    def fire_gather(k, r):
        off = base + k * _B
        pltpu.async_copy(t2_hbm.at[idx[r]], g2[r], semg[r])
        pltpu.async_copy(ea_hbm.at[pl.ds(off, _B)], eav[r], seme[r])

    def wait_gather(k, r):
        off = base + k * _B
        pltpu.make_async_copy(t2_hbm.at[idx[r]], g2[r], semg[r]).wait()
        pltpu.make_async_copy(ea_hbm.at[pl.ds(off, _B)], eav[r],
                              seme[r]).wait()

    def fire_scatter(r):
        pltpu.async_copy(msg[r], agg.at[dsts[r]], sems[r], add=True)

    def wait_scatter(r):
        pltpu.make_async_copy(msg[r], agg.at[dsts[r]], sems[r]).wait()

    def copy_dsts(r):
        dsts[r][pl.ds(0, 16)] = idx[r][pl.ds(0, 16)]
        dsts[r][pl.ds(16, 16)] = idx[r][pl.ds(16, 16)]
        dsts[r][pl.ds(24, 16)] = idx[r][pl.ds(24, 16)]

    def compute(r):
        @pl.loop(0, _B)
        def _edge(i):
            for g in range(_F // 32):
                cf = 32 * g
                wf = 16 * g
                ws = _F // 2 + 16 * g
                wdf = _w16(g2[r], i, wf)
                wsf = _w16(g2[r], _B + i, wf)
                wef = _w16(eav[r], i, wf)
                wds = _w16(g2[r], i, ws)
                wss = _w16(g2[r], _B + i, ws)
                wes = _w16(eav[r], i, ws)
                for h, ext in enumerate((_lo_f32, _hi_f32)):
                    zf = ext(wdf) + ext(wsf) + ext(wef)
                    zs = ext(wds) + ext(wss) + ext(wes)
                    gate = 1.0 / (1.0 + jnp.exp(-zf))
                    msg[r][i, pl.ds(cf + 16 * h, 16)] = gate * _softplus(zs)

    fire_idx(0, 0)
    fire_idx(1, 1)

    @pl.loop(0, _ZROWS)
    def _zero(i):
        for j in range(_F // 16):
            zbuf[i, pl.ds(j * 16, 16)] = jnp.zeros((16,), jnp.float32)

    for t in range(_ROWS_W // _ZROWS):
        pltpu.sync_copy(zbuf, agg.at[pl.ds(s * _ROWS_W + t * _ZROWS, _ZROWS)])

    @pl.when(s == 0)
    def _zero_tail():
        pltpu.sync_copy(zbuf.at[pl.ds(0, _TAIL)],
                        agg.at[pl.ds(_NS * _ROWS_W, _TAIL)])

    wait_idx(0, 0)
    fire_gather(0, 0)
    plsc.subcore_barrier()

    wait_idx(1, 1)
    fire_gather(1, 1)
    wait_gather(0, 0)
    copy_dsts(0)
    fire_idx(2, 0)
    compute(0)
    fire_scatter(0)

    wait_idx(2, 0)
    fire_gather(2, 0)
    wait_gather(1, 1)
    copy_dsts(1)
    fire_idx(3, 1)
    compute(1)
    fire_scatter(1)

    @pl.loop(0, (_CHUNKS - 4) // 2)
    def _pair(m):
        k = 2 * m + 2
        for p in range(2):
            r = p
            rn = 1 - p
            wait_idx(k + p + 1, rn)
            fire_gather(k + p + 1, rn)
            wait_gather(k + p, r)
            wait_scatter(r)
            copy_dsts(r)
            fire_idx(k + p + 2, r)
            compute(r)
            fire_scatter(r)

    wait_idx(_CHUNKS - 1, 1)
    fire_gather(_CHUNKS - 1, 1)
    wait_gather(_CHUNKS - 2, 0)
    wait_scatter(0)
    copy_dsts(0)
    compute(0)
    fire_scatter(0)

    wait_gather(_CHUNKS - 1, 1)
    wait_scatter(1)
    copy_dsts(1)
    compute(1)
    fire_scatter(1)

    wait_scatter(0)
    wait_scatter(1)
    plsc.subcore_barrier()

    pltpu.sync_copy(agg.at[pl.ds(s * _ROWS_W, _ROWS_W)],
                    out_hbm.at[c, pl.ds(s * _ROWS_W, _ROWS_W)])

    @pl.when(s == 0)
    def _write_tail():
        pltpu.sync_copy(agg.at[pl.ds(_NS * _ROWS_W, _TAIL)],
                        out_hbm.at[c, pl.ds(_NS * _ROWS_W, _TAIL)])


def _sc_edge(T2, EA, dst, srcp):
    mesh = plsc.VectorSubcoreMesh(
        core_axis_name="c", subcore_axis_name="s",
        num_cores=_NC, num_subcores=_NS)
    cp = pltpu.CompilerParams()
    if "needs_layout_passes" in pltpu.CompilerParams.__dataclass_fields__:
        cp = dataclasses.replace(cp, needs_layout_passes=False)
    fn = pl.kernel(
        _sc_edge_body,
        out_type=jax.ShapeDtypeStruct((_NC, _N, _F), jnp.float32),
        mesh=mesh,
        compiler_params=cp,
        scratch_types=[
            [pltpu.VMEM((2 * _B,), jnp.int32)] * 2,
            [pltpu.VMEM((_B,), jnp.int32)] * 2,
            [pltpu.VMEM((2 * _B, _F), jnp.int32)] * 2,
            [pltpu.VMEM((_B, _F), jnp.int32)] * 2,
            [pltpu.VMEM((_B, _F), jnp.float32)] * 2,
            pltpu.VMEM((_ZROWS, _F), jnp.float32),
            pltpu.VMEM_SHARED((_N, _F), jnp.float32),
            [pltpu.SemaphoreType.DMA] * 2,
            [pltpu.SemaphoreType.DMA] * 2,
            [pltpu.SemaphoreType.DMA] * 2,
            [pltpu.SemaphoreType.DMA] * 2,
        ],
    )
    return fn(T2, EA, dst, srcp)


def kernel(x, edge_index, edge_attr, batch, Wf0, bf0, Ws0, bs0,
           Wf1, bf1, Ws1, bs1, gamma0, beta0, gamma1, beta1, Wfc, bfc):
    src = edge_index[0].astype(jnp.int32)
    dst = edge_index[1].astype(jnp.int32)
    srcp = src + _N
    seg = batch.astype(jnp.int32)

    lop = jnp.array(_LOPERM, dtype=jnp.int32)
    hip = jnp.array(_HIPERM, dtype=jnp.int32)

    def split(Wf, Ws, bf, bs):
        W = jnp.concatenate([Wf, Ws], axis=1)
        b = jnp.concatenate([bf, bs])
        Wd, Wsrc, We = W[:_F], W[_F:2 * _F], W[2 * _F:]
        return ((Wd[:, lop], Wd[:, hip]), (Wsrc[:, lop], Wsrc[:, hip]),
                (We[:, lop], We[:, hip]), (b[lop], b[hip]))

    Wd0, Wsrc0, We0, b0 = split(Wf0, Ws0, bf0, bs0)
    Wd1, Wsrc1, We1, b1 = split(Wf1, Ws1, bf1, bs1)

    EA0 = _edge_terms(edge_attr, We0[0], b0[0], We0[1], b0[1])
    EA1 = _edge_terms(edge_attr, We1[0], b1[0], We1[1], b1[1])

    T20 = _tables(x, jnp.stack([Wd0[0], Wsrc0[0]]),
                  jnp.stack([Wd0[1], Wsrc0[1]]))
    parts0 = _sc_edge(T20, EA0, dst, srcp)

    u0, s0, q0 = _stats(x, parts0)
    h1, T21 = _mid(u0, s0, q0, gamma0, beta0,
                   jnp.stack([Wd1[0], Wsrc1[0]]), jnp.stack([Wd1[1], Wsrc1[1]]))
    parts1 = _sc_edge(T21, EA1, dst, srcp)

    u1, s1, q1 = _stats(h1, parts1)
    return _final(u1, s1, q1, gamma1, beta1, seg, Wfc, bfc)


# speedup vs baseline: 5.7066x; 1.0675x over previous
import dataclasses
import functools

import jax
import jax.numpy as jnp
from jax import lax
from jax.experimental import pallas as pl
from jax.experimental.pallas import tpu as pltpu
from jax.experimental.pallas import tpu_sc as plsc

_N = 10000
_E = 320000
_F = 128
_D = 16
_G = 64
_C = 16
_EPS = 1e-5

_NC = 2
_NS = 16
_NW = _NC * _NS
_PER_W = _E // _NW
_B = 40
_CHUNKS = _PER_W // _B
_ROWS_W = 624
_TAIL = _N - _NS * _ROWS_W
_ZROWS = 48

_Q = (0.9999999705990521, -0.4999950177195202, 0.3331927456296667,
      -0.24844387923813097, 0.19111483813148478, -0.1367485547842574,
      0.07836244160823844, -0.029588863662033647, 0.0052535214410623764)

_PREC = lax.Precision.HIGHEST


def _dot(a, b):
    return jnp.dot(a, b, preferred_element_type=jnp.float32, precision=_PREC)


_LOPERM = tuple(32 * g + t for g in range(8) for t in range(16))
_HIPERM = tuple(32 * g + 16 + t for g in range(8) for t in range(16))


def _pack2(lo, hi):
    lob = lax.bitcast_convert_type(
        lo.astype(jnp.bfloat16).astype(jnp.float32), jnp.int32)
    hib = lax.bitcast_convert_type(
        hi.astype(jnp.bfloat16).astype(jnp.float32), jnp.int32)
    return jnp.bitwise_or(jnp.bitwise_and(hib, jnp.int32(-65536)),
                          lax.shift_right_logical(lob, 16))


def _edge_terms_body(ea_ref, wlo_ref, blo_ref, whi_ref, bhi_ref, o_ref):
    ea = ea_ref[...]
    o_ref[...] = _pack2(_dot(ea, wlo_ref[...]) + blo_ref[...],
                        _dot(ea, whi_ref[...]) + bhi_ref[...])


def _edge_terms(edge_attr, Wlo, blo, Whi, bhi):
    eb = 8000
    return pl.pallas_call(
        _edge_terms_body,
        grid=(_E // eb,),
        in_specs=[
            pl.BlockSpec((eb, _D), lambda i: (i, 0)),
            pl.BlockSpec((_D, _F), lambda i: (0, 0)),
            pl.BlockSpec((1, _F), lambda i: (0, 0)),
            pl.BlockSpec((_D, _F), lambda i: (0, 0)),
            pl.BlockSpec((1, _F), lambda i: (0, 0)),
        ],
        out_specs=pl.BlockSpec((eb, _F), lambda i: (i, 0)),
        out_shape=jax.ShapeDtypeStruct((_E, _F), jnp.int32),
    )(edge_attr, Wlo, blo.reshape(1, -1), Whi, bhi.reshape(1, -1))


_TRB = 2000


def _tables_body(x_ref, wlo_ref, whi_ref, t2_ref):
    xv = x_ref[...]
    t2_ref[...] = _pack2(_dot(xv, wlo_ref[0]), _dot(xv, whi_ref[0]))


def _tables(h, W2lo, W2hi):
    nrb = _N // _TRB
    return pl.pallas_call(
        _tables_body,
        grid=(2, nrb),
        in_specs=[
            pl.BlockSpec((_TRB, _F), lambda t, i: (i, 0)),
            pl.BlockSpec((1, _F, _F), lambda t, i: (t, 0, 0)),
            pl.BlockSpec((1, _F, _F), lambda t, i: (t, 0, 0)),
        ],
        out_specs=pl.BlockSpec((_TRB, _F), lambda t, i: (t * nrb + i, 0)),
        out_shape=jax.ShapeDtypeStruct((2 * _N, _F), jnp.int32),
    )(h, W2lo, W2hi)


_RB = 2000
_NRB = _N // _RB


def _stats_body(x_ref, p_ref, u_ref, sum_ref, sq_ref):
    u = x_ref[...] + p_ref[0] + p_ref[1]
    u_ref[...] = u
    ps = jnp.sum(u, axis=0, keepdims=True)
    pq = jnp.sum(u * u, axis=0, keepdims=True)

    @pl.when(pl.program_id(0) == 0)
    def _init():
        sum_ref[...] = ps
        sq_ref[...] = pq

    @pl.when(pl.program_id(0) > 0)
    def _acc():
        sum_ref[...] += ps
        sq_ref[...] += pq


def _stats(x, parts):
    return pl.pallas_call(
        _stats_body,
        grid=(_NRB,),
        in_specs=[
            pl.BlockSpec((_RB, _F), lambda i: (i, 0)),
            pl.BlockSpec((2, _RB, _F), lambda i: (0, i, 0)),
        ],
        out_specs=[
            pl.BlockSpec((_RB, _F), lambda i: (i, 0)),
            pl.BlockSpec((1, _F), lambda i: (0, 0)),
            pl.BlockSpec((1, _F), lambda i: (0, 0)),
        ],
        out_shape=[
            jax.ShapeDtypeStruct((_N, _F), jnp.float32),
            jax.ShapeDtypeStruct((1, _F), jnp.float32),
            jax.ShapeDtypeStruct((1, _F), jnp.float32),
        ],
    )(x, parts)


def _bn_from_stats(u, s, q, gamma, beta):
    m = s / _N
    v = q / _N - m * m
    return (u - m) / jnp.sqrt(v + _EPS) * gamma + beta


def _mid_body(u_ref, s_ref, q_ref, g_ref, b_ref, wlo_ref, whi_ref,
              h_ref, t2_ref):
    h = _bn_from_stats(u_ref[...], s_ref[...], q_ref[...],
                       g_ref[...], b_ref[...])
    h_ref[...] = h
    t2_ref[...] = _pack2(_dot(h, wlo_ref[0]), _dot(h, whi_ref[0]))


def _mid(u, s, q, gamma, beta, W2lo, W2hi):
    nrb = _N // _RB
    return pl.pallas_call(
        _mid_body,
        grid=(2, nrb),
        in_specs=[
            pl.BlockSpec((_RB, _F), lambda t, i: (i, 0)),
            pl.BlockSpec((1, _F), lambda t, i: (0, 0)),
            pl.BlockSpec((1, _F), lambda t, i: (0, 0)),
            pl.BlockSpec((1, _F), lambda t, i: (0, 0)),
            pl.BlockSpec((1, _F), lambda t, i: (0, 0)),
            pl.BlockSpec((1, _F, _F), lambda t, i: (t, 0, 0)),
            pl.BlockSpec((1, _F, _F), lambda t, i: (t, 0, 0)),
        ],
        out_specs=[
            pl.BlockSpec((_RB, _F), lambda t, i: (i, 0)),
            pl.BlockSpec((_RB, _F), lambda t, i: (t * nrb + i, 0)),
        ],
        out_shape=[
            jax.ShapeDtypeStruct((_N, _F), jnp.float32),
            jax.ShapeDtypeStruct((2 * _N, _F), jnp.int32),
        ],
    )(u, s, q, gamma.reshape(1, -1), beta.reshape(1, -1), W2lo, W2hi)


def _final_body(u_ref, s_ref, q_ref, g_ref, b_ref, seg_ref, wfc_ref, bfc_ref,
                o_ref, sacc_ref, cacc_ref):
    h2 = _bn_from_stats(u_ref[...], s_ref[...], q_ref[...],
                        g_ref[...], b_ref[...])
    onehot = (seg_ref[...] == lax.broadcasted_iota(jnp.int32, (_RB, _G), 1)
              ).astype(jnp.float32)
    ps = lax.dot_general(onehot, h2, (((0,), (0,)), ((), ())),
                         precision=_PREC, preferred_element_type=jnp.float32)
    pc = jnp.sum(onehot, axis=0)[:, None]
    i = pl.program_id(0)

    @pl.when(i == 0)
    def _init():
        sacc_ref[...] = ps
        cacc_ref[...] = pc

    @pl.when(i > 0)
    def _acc():
        sacc_ref[...] += ps
        cacc_ref[...] += pc

    @pl.when(i == _NRB - 1)
    def _emit():
        pooled = sacc_ref[...] / jnp.clip(cacc_ref[...], 1.0, None)
        o_ref[...] = _dot(pooled, wfc_ref[...]) + bfc_ref[...]


def _final(u, s, q, gamma, beta, seg, Wfc, bfc):
    return pl.pallas_call(
        _final_body,
        grid=(_NRB,),
        in_specs=[
            pl.BlockSpec((_RB, _F), lambda i: (i, 0)),
            pl.BlockSpec((1, _F), lambda i: (0, 0)),
            pl.BlockSpec((1, _F), lambda i: (0, 0)),
            pl.BlockSpec((1, _F), lambda i: (0, 0)),
            pl.BlockSpec((1, _F), lambda i: (0, 0)),
            pl.BlockSpec((_RB, 1), lambda i: (i, 0)),
            pl.BlockSpec((_F, _C), lambda i: (0, 0)),
            pl.BlockSpec((1, _C), lambda i: (0, 0)),
        ],
        out_specs=pl.BlockSpec((_G, _C), lambda i: (0, 0)),
        out_shape=jax.ShapeDtypeStruct((_G, _C), jnp.float32),
        scratch_shapes=[
            pltpu.VMEM((_G, _F), jnp.float32),
            pltpu.VMEM((_G, 1), jnp.float32),
        ],
    )(u, s, q, gamma.reshape(1, -1), beta.reshape(1, -1),
      seg.reshape(_N, 1), Wfc, bfc.reshape(1, -1))


def _softplus(x):
    u = jnp.exp(-jnp.abs(x))
    q = jnp.float32(_Q[-1])
    for c in _Q[-2::-1]:
        q = q * u + jnp.float32(c)
    return jnp.maximum(x, 0.0) + q * u


def _lo_f32(w):
    return plsc.bitcast(jnp.left_shift(w, 16), jnp.float32)


def _hi_f32(w):
    return plsc.bitcast(jnp.bitwise_and(w, jnp.int32(-65536)), jnp.float32)


def _w16(ref, i, wcol):
    return ref[i, pl.ds(wcol, 16)]


def _sc_edge_body(t2_hbm, ea_hbm, ei_hbm, out_hbm,
                  idx, dsts, g2, eav, msg, zbuf, agg, semi, semg, seme, sems):
    c = lax.axis_index("c")
    s = lax.axis_index("s")
    base = (c * _NS + s) * _PER_W

    def fire_idx(k, r):
        off = base + k * _B
        pltpu.async_copy(ei_hbm.at[pl.ds(_E + off, _B)],
                         idx[r].at[pl.ds(0, _B)], semi[r])
        pltpu.async_copy(ei_hbm.at[pl.ds(off, _B)],
                         idx[r].at[pl.ds(_B, _B)], semi[r])

    def wait_idx(k, r):
        off = base + k * _B
        pltpu.make_async_copy(ei_hbm.at[pl.ds(_E + off, _B)],
                              idx[r].at[pl.ds(0, _B)], semi[r]).wait()
        pltpu.make_async_copy(ei_hbm.at[pl.ds(off, _B)],
                              idx[r].at[pl.ds(_B, _B)], semi[r]).wait()
        for o in (_B, _B + 16, _B + 32):
            idx[r][pl.ds(o, 16)] = idx[r][pl.ds(o, 16)] + _N

    def fire_gather(k, r):
        off = base + k * _B
        pltpu.async_copy(t2_hbm.at[idx[r].at[pl.ds(0, 2 * _B)]],
                         g2[r], semg[r])
        pltpu.async_copy(ea_hbm.at[pl.ds(off, _B)], eav[r], seme[r])

    def wait_gather(k, r):
        off = base + k * _B
        pltpu.make_async_copy(t2_hbm.at[idx[r].at[pl.ds(0, 2 * _B)]],
                              g2[r], semg[r]).wait()
        pltpu.make_async_copy(ea_hbm.at[pl.ds(off, _B)], eav[r],
                              seme[r]).wait()

    def fire_scatter(r):
        pltpu.async_copy(msg[r], agg.at[dsts[r]], sems[r], add=True)

    def wait_scatter(r):
        pltpu.make_async_copy(msg[r], agg.at[dsts[r]], sems[r]).wait()

    def copy_dsts(r):
        dsts[r][pl.ds(0, 16)] = idx[r][pl.ds(0, 16)]
        dsts[r][pl.ds(16, 16)] = idx[r][pl.ds(16, 16)]
        dsts[r][pl.ds(24, 16)] = idx[r][pl.ds(24, 16)]

    def compute(r):
        @pl.loop(0, _B)
        def _edge(i):
            for g in range(_F // 32):
                cf = 32 * g
                wf = 16 * g
                ws = _F // 2 + 16 * g
                wdf = _w16(g2[r], i, wf)
                wsf = _w16(g2[r], _B + i, wf)
                wef = _w16(eav[r], i, wf)
                wds = _w16(g2[r], i, ws)
                wss = _w16(g2[r], _B + i, ws)
                wes = _w16(eav[r], i, ws)
                for h, ext in enumerate((_lo_f32, _hi_f32)):
                    zf = ext(wdf) + ext(wsf) + ext(wef)
                    zs = ext(wds) + ext(wss) + ext(wes)
                    gate = 1.0 / (1.0 + jnp.exp(-zf))
                    msg[r][i, pl.ds(cf + 16 * h, 16)] = gate * _softplus(zs)

    fire_idx(0, 0)
    fire_idx(1, 1)

    @pl.loop(0, _ZROWS)
    def _zero(i):
        for j in range(_F // 16):
            zbuf[i, pl.ds(j * 16, 16)] = jnp.zeros((16,), jnp.float32)

    for t in range(_ROWS_W // _ZROWS):
        pltpu.sync_copy(zbuf, agg.at[pl.ds(s * _ROWS_W + t * _ZROWS, _ZROWS)])

    @pl.when(s == 0)
    def _zero_tail():
        pltpu.sync_copy(zbuf.at[pl.ds(0, _TAIL)],
                        agg.at[pl.ds(_NS * _ROWS_W, _TAIL)])

    wait_idx(0, 0)
    fire_gather(0, 0)
    plsc.subcore_barrier()

    wait_idx(1, 1)
    fire_gather(1, 1)
    wait_gather(0, 0)
    copy_dsts(0)
    fire_idx(2, 0)
    compute(0)
    fire_scatter(0)

    wait_idx(2, 0)
    fire_gather(2, 0)
    wait_gather(1, 1)
    copy_dsts(1)
    fire_idx(3, 1)
    compute(1)
    fire_scatter(1)

    @pl.loop(0, (_CHUNKS - 4) // 2)
    def _pair(m):
        k = 2 * m + 2
        for p in range(2):
            r = p
            rn = 1 - p
            wait_idx(k + p + 1, rn)
            fire_gather(k + p + 1, rn)
            wait_gather(k + p, r)
            wait_scatter(r)
            copy_dsts(r)
            fire_idx(k + p + 2, r)
            compute(r)
            fire_scatter(r)

    wait_idx(_CHUNKS - 1, 1)
    fire_gather(_CHUNKS - 1, 1)
    wait_gather(_CHUNKS - 2, 0)
    wait_scatter(0)
    copy_dsts(0)
    compute(0)
    fire_scatter(0)

    wait_gather(_CHUNKS - 1, 1)
    wait_scatter(1)
    copy_dsts(1)
    compute(1)
    fire_scatter(1)

    wait_scatter(0)
    wait_scatter(1)
    plsc.subcore_barrier()

    pltpu.sync_copy(agg.at[pl.ds(s * _ROWS_W, _ROWS_W)],
                    out_hbm.at[c, pl.ds(s * _ROWS_W, _ROWS_W)])

    @pl.when(s == 0)
    def _write_tail():
        pltpu.sync_copy(agg.at[pl.ds(_NS * _ROWS_W, _TAIL)],
                        out_hbm.at[c, pl.ds(_NS * _ROWS_W, _TAIL)])


def _sc_edge(T2, EA, ei):
    mesh = plsc.VectorSubcoreMesh(
        core_axis_name="c", subcore_axis_name="s",
        num_cores=_NC, num_subcores=_NS)
    cp = pltpu.CompilerParams()
    if "needs_layout_passes" in pltpu.CompilerParams.__dataclass_fields__:
        cp = dataclasses.replace(cp, needs_layout_passes=False)
    fn = pl.kernel(
        _sc_edge_body,
        out_type=jax.ShapeDtypeStruct((_NC, _N, _F), jnp.float32),
        mesh=mesh,
        compiler_params=cp,
        scratch_types=[
            [pltpu.VMEM((2 * _B + 16,), jnp.int32)] * 2,
            [pltpu.VMEM((_B,), jnp.int32)] * 2,
            [pltpu.VMEM((2 * _B, _F), jnp.int32)] * 2,
            [pltpu.VMEM((_B, _F), jnp.int32)] * 2,
            [pltpu.VMEM((_B, _F), jnp.float32)] * 2,
            pltpu.VMEM((_ZROWS, _F), jnp.float32),
            pltpu.VMEM_SHARED((_N, _F), jnp.float32),
            [pltpu.SemaphoreType.DMA] * 2,
            [pltpu.SemaphoreType.DMA] * 2,
            [pltpu.SemaphoreType.DMA] * 2,
            [pltpu.SemaphoreType.DMA] * 2,
        ],
    )
    return fn(T2, EA, ei)


def kernel(x, edge_index, edge_attr, batch, Wf0, bf0, Ws0, bs0,
           Wf1, bf1, Ws1, bs1, gamma0, beta0, gamma1, beta1, Wfc, bfc):
    ei = edge_index.astype(jnp.int32).reshape(2 * _E)
    seg = batch.astype(jnp.int32)

    lop = jnp.array(_LOPERM, dtype=jnp.int32)
    hip = jnp.array(_HIPERM, dtype=jnp.int32)

    def split(Wf, Ws, bf, bs):
        W = jnp.concatenate([Wf, Ws], axis=1)
        b = jnp.concatenate([bf, bs])
        Wd, Wsrc, We = W[:_F], W[_F:2 * _F], W[2 * _F:]
        return ((Wd[:, lop], Wd[:, hip]), (Wsrc[:, lop], Wsrc[:, hip]),
                (We[:, lop], We[:, hip]), (b[lop], b[hip]))

    Wd0, Wsrc0, We0, b0 = split(Wf0, Ws0, bf0, bs0)
    Wd1, Wsrc1, We1, b1 = split(Wf1, Ws1, bf1, bs1)

    EA0 = _edge_terms(edge_attr, We0[0], b0[0], We0[1], b0[1])
    EA1 = _edge_terms(edge_attr, We1[0], b1[0], We1[1], b1[1])

    T20 = _tables(x, jnp.stack([Wd0[0], Wsrc0[0]]),
                  jnp.stack([Wd0[1], Wsrc0[1]]))
    parts0 = _sc_edge(T20, EA0, ei)

    u0, s0, q0 = _stats(x, parts0)
    h1, T21 = _mid(u0, s0, q0, gamma0, beta0,
                   jnp.stack([Wd1[0], Wsrc1[0]]), jnp.stack([Wd1[1], Wsrc1[1]]))
    parts1 = _sc_edge(T21, EA1, ei)

    u1, s1, q1 = _stats(h1, parts1)
    return _final(u1, s1, q1, gamma1, beta1, seg, Wfc, bfc)


# speedup vs baseline: 6.2708x; 1.0989x over previous
import dataclasses
import functools

import jax
import jax.numpy as jnp
from jax import lax
from jax.experimental import pallas as pl
from jax.experimental.pallas import tpu as pltpu
from jax.experimental.pallas import tpu_sc as plsc

_N = 10000
_E = 320000
_F = 128
_D = 16
_G = 64
_C = 16
_EPS = 1e-5

_NC = 2
_NS = 16
_NW = _NC * _NS
_PER_W = _E // _NW
_B = 40
_CHUNKS = _PER_W // _B
_ROWS_W = 624
_TAIL = _N - _NS * _ROWS_W
_ZROWS = 48

_Q = (0.9999999705990521, -0.4999950177195202, 0.3331927456296667,
      -0.24844387923813097, 0.19111483813148478, -0.1367485547842574,
      0.07836244160823844, -0.029588863662033647, 0.0052535214410623764)

_PREC = lax.Precision.HIGHEST


def _dot(a, b):
    return jnp.dot(a, b, preferred_element_type=jnp.float32, precision=_PREC)


_LOPERM = tuple(32 * g + t for g in range(8) for t in range(16))
_HIPERM = tuple(32 * g + 16 + t for g in range(8) for t in range(16))


def _pack2(lo, hi):
    lob = lax.bitcast_convert_type(
        lo.astype(jnp.bfloat16).astype(jnp.float32), jnp.int32)
    hib = lax.bitcast_convert_type(
        hi.astype(jnp.bfloat16).astype(jnp.float32), jnp.int32)
    return jnp.bitwise_or(jnp.bitwise_and(hib, jnp.int32(-65536)),
                          lax.shift_right_logical(lob, 16))


def _edge_terms_body(ea_ref, wlo_ref, blo_ref, whi_ref, bhi_ref, o_ref):
    ea = ea_ref[...]
    lo = jnp.dot(ea, wlo_ref[...], preferred_element_type=jnp.float32)
    hi = jnp.dot(ea, whi_ref[...], preferred_element_type=jnp.float32)
    o_ref[...] = _pack2(lo + blo_ref[...], hi + bhi_ref[...])


def _edge_terms(edge_attr, Wlo, blo, Whi, bhi):
    eb = 8000
    return pl.pallas_call(
        _edge_terms_body,
        grid=(_E // eb,),
        in_specs=[
            pl.BlockSpec((eb, _D), lambda i: (i, 0)),
            pl.BlockSpec((_D, _F), lambda i: (0, 0)),
            pl.BlockSpec((1, _F), lambda i: (0, 0)),
            pl.BlockSpec((_D, _F), lambda i: (0, 0)),
            pl.BlockSpec((1, _F), lambda i: (0, 0)),
        ],
        out_specs=pl.BlockSpec((eb, _F), lambda i: (i, 0)),
        out_shape=jax.ShapeDtypeStruct((_E, _F), jnp.int32),
    )(edge_attr, Wlo, blo.reshape(1, -1), Whi, bhi.reshape(1, -1))


_TRB = 2000


def _tables_body(x_ref, wlo_ref, whi_ref, t2_ref):
    xv = x_ref[...]
    t2_ref[...] = _pack2(_dot(xv, wlo_ref[0]), _dot(xv, whi_ref[0]))


def _tables(h, W2lo, W2hi):
    nrb = _N // _TRB
    return pl.pallas_call(
        _tables_body,
        grid=(2, nrb),
        in_specs=[
            pl.BlockSpec((_TRB, _F), lambda t, i: (i, 0)),
            pl.BlockSpec((1, _F, _F), lambda t, i: (t, 0, 0)),
            pl.BlockSpec((1, _F, _F), lambda t, i: (t, 0, 0)),
        ],
        out_specs=pl.BlockSpec((_TRB, _F), lambda t, i: (t * nrb + i, 0)),
        out_shape=jax.ShapeDtypeStruct((2 * _N, _F), jnp.int32),
    )(h, W2lo, W2hi)


_RB = 2000
_NRB = _N // _RB


def _stats_body(x_ref, p_ref, u_ref, sum_ref, sq_ref):
    u = x_ref[...] + p_ref[0] + p_ref[1]
    u_ref[...] = u
    ps = jnp.sum(u, axis=0, keepdims=True)
    pq = jnp.sum(u * u, axis=0, keepdims=True)

    @pl.when(pl.program_id(0) == 0)
    def _init():
        sum_ref[...] = ps
        sq_ref[...] = pq

    @pl.when(pl.program_id(0) > 0)
    def _acc():
        sum_ref[...] += ps
        sq_ref[...] += pq


def _stats(x, parts):
    return pl.pallas_call(
        _stats_body,
        grid=(_NRB,),
        in_specs=[
            pl.BlockSpec((_RB, _F), lambda i: (i, 0)),
            pl.BlockSpec((2, _RB, _F), lambda i: (0, i, 0)),
        ],
        out_specs=[
            pl.BlockSpec((_RB, _F), lambda i: (i, 0)),
            pl.BlockSpec((1, _F), lambda i: (0, 0)),
            pl.BlockSpec((1, _F), lambda i: (0, 0)),
        ],
        out_shape=[
            jax.ShapeDtypeStruct((_N, _F), jnp.float32),
            jax.ShapeDtypeStruct((1, _F), jnp.float32),
            jax.ShapeDtypeStruct((1, _F), jnp.float32),
        ],
    )(x, parts)


def _bn_from_stats(u, s, q, gamma, beta):
    m = s / _N
    v = q / _N - m * m
    return (u - m) / jnp.sqrt(v + _EPS) * gamma + beta


def _mid_body(u_ref, s_ref, q_ref, g_ref, b_ref, wlo_ref, whi_ref,
              h_ref, t2_ref):
    h = _bn_from_stats(u_ref[...], s_ref[...], q_ref[...],
                       g_ref[...], b_ref[...])
    h_ref[...] = h
    t2_ref[...] = _pack2(_dot(h, wlo_ref[0]), _dot(h, whi_ref[0]))


def _mid(u, s, q, gamma, beta, W2lo, W2hi):
    nrb = _N // _RB
    return pl.pallas_call(
        _mid_body,
        grid=(2, nrb),
        in_specs=[
            pl.BlockSpec((_RB, _F), lambda t, i: (i, 0)),
            pl.BlockSpec((1, _F), lambda t, i: (0, 0)),
            pl.BlockSpec((1, _F), lambda t, i: (0, 0)),
            pl.BlockSpec((1, _F), lambda t, i: (0, 0)),
            pl.BlockSpec((1, _F), lambda t, i: (0, 0)),
            pl.BlockSpec((1, _F, _F), lambda t, i: (t, 0, 0)),
            pl.BlockSpec((1, _F, _F), lambda t, i: (t, 0, 0)),
        ],
        out_specs=[
            pl.BlockSpec((_RB, _F), lambda t, i: (i, 0)),
            pl.BlockSpec((_RB, _F), lambda t, i: (t * nrb + i, 0)),
        ],
        out_shape=[
            jax.ShapeDtypeStruct((_N, _F), jnp.float32),
            jax.ShapeDtypeStruct((2 * _N, _F), jnp.int32),
        ],
    )(u, s, q, gamma.reshape(1, -1), beta.reshape(1, -1), W2lo, W2hi)


def _final_body(u_ref, s_ref, q_ref, g_ref, b_ref, seg_ref, wfc_ref, bfc_ref,
                o_ref, sacc_ref, cacc_ref):
    h2 = _bn_from_stats(u_ref[...], s_ref[...], q_ref[...],
                        g_ref[...], b_ref[...])
    onehot = (seg_ref[...] == lax.broadcasted_iota(jnp.int32, (_RB, _G), 1)
              ).astype(jnp.float32)
    ps = lax.dot_general(onehot, h2, (((0,), (0,)), ((), ())),
                         precision=_PREC, preferred_element_type=jnp.float32)
    pc = jnp.sum(onehot, axis=0)[:, None]
    i = pl.program_id(0)

    @pl.when(i == 0)
    def _init():
        sacc_ref[...] = ps
        cacc_ref[...] = pc

    @pl.when(i > 0)
    def _acc():
        sacc_ref[...] += ps
        cacc_ref[...] += pc

    @pl.when(i == _NRB - 1)
    def _emit():
        pooled = sacc_ref[...] / jnp.clip(cacc_ref[...], 1.0, None)
        o_ref[...] = _dot(pooled, wfc_ref[...]) + bfc_ref[...]


def _final(u, s, q, gamma, beta, seg, Wfc, bfc):
    return pl.pallas_call(
        _final_body,
        grid=(_NRB,),
        in_specs=[
            pl.BlockSpec((_RB, _F), lambda i: (i, 0)),
            pl.BlockSpec((1, _F), lambda i: (0, 0)),
            pl.BlockSpec((1, _F), lambda i: (0, 0)),
            pl.BlockSpec((1, _F), lambda i: (0, 0)),
            pl.BlockSpec((1, _F), lambda i: (0, 0)),
            pl.BlockSpec((_RB, 1), lambda i: (i, 0)),
            pl.BlockSpec((_F, _C), lambda i: (0, 0)),
            pl.BlockSpec((1, _C), lambda i: (0, 0)),
        ],
        out_specs=pl.BlockSpec((_G, _C), lambda i: (0, 0)),
        out_shape=jax.ShapeDtypeStruct((_G, _C), jnp.float32),
        scratch_shapes=[
            pltpu.VMEM((_G, _F), jnp.float32),
            pltpu.VMEM((_G, 1), jnp.float32),
        ],
    )(u, s, q, gamma.reshape(1, -1), beta.reshape(1, -1),
      seg.reshape(_N, 1), Wfc, bfc.reshape(1, -1))


def _softplus(x):
    u = jnp.exp(-jnp.abs(x))
    q = jnp.float32(_Q[-1])
    for c in _Q[-2::-1]:
        q = q * u + jnp.float32(c)
    return jnp.maximum(x, 0.0) + q * u


def _lo_f32(w):
    return plsc.bitcast(jnp.left_shift(w, 16), jnp.float32)


def _hi_f32(w):
    return plsc.bitcast(jnp.bitwise_and(w, jnp.int32(-65536)), jnp.float32)


def _w16(ref, i, wcol):
    return ref[i, pl.ds(wcol, 16)]


def _sc_edge_body(t2_hbm, ea_hbm, ei_hbm, out_hbm,
                  idx, dsts, g2, eav, msg, zbuf, agg, semi, semg, seme, sems):
    c = lax.axis_index("c")
    s = lax.axis_index("s")
    base = (c * _NS + s) * _PER_W

    def fire_idx(k, r):
        off = base + k * _B
        pltpu.async_copy(ei_hbm.at[pl.ds(_E + off, _B)],
                         idx[r].at[pl.ds(0, _B)], semi[r])
        pltpu.async_copy(ei_hbm.at[pl.ds(off, _B)],
                         idx[r].at[pl.ds(_B, _B)], semi[r])

    def wait_idx(k, r):
        off = base + k * _B
        pltpu.make_async_copy(ei_hbm.at[pl.ds(_E + off, _B)],
                              idx[r].at[pl.ds(0, _B)], semi[r]).wait()
        pltpu.make_async_copy(ei_hbm.at[pl.ds(off, _B)],
                              idx[r].at[pl.ds(_B, _B)], semi[r]).wait()
        for o in (_B, _B + 16, _B + 32):
            idx[r][pl.ds(o, 16)] = idx[r][pl.ds(o, 16)] + _N

    def fire_gather(k, r):
        off = base + k * _B
        pltpu.async_copy(t2_hbm.at[idx[r].at[pl.ds(0, 2 * _B)]],
                         g2[r], semg[r])
        pltpu.async_copy(ea_hbm.at[pl.ds(off, _B)], eav[r], seme[r])

    def wait_gather(k, r):
        off = base + k * _B
        pltpu.make_async_copy(t2_hbm.at[idx[r].at[pl.ds(0, 2 * _B)]],
                              g2[r], semg[r]).wait()
        pltpu.make_async_copy(ea_hbm.at[pl.ds(off, _B)], eav[r],
                              seme[r]).wait()

    def fire_scatter(r):
        pltpu.async_copy(msg[r], agg.at[dsts[r]], sems[r], add=True)

    def wait_scatter(r):
        pltpu.make_async_copy(msg[r], agg.at[dsts[r]], sems[r]).wait()

    def copy_dsts(r):
        dsts[r][pl.ds(0, 16)] = idx[r][pl.ds(0, 16)]
        dsts[r][pl.ds(16, 16)] = idx[r][pl.ds(16, 16)]
        dsts[r][pl.ds(24, 16)] = idx[r][pl.ds(24, 16)]

    def compute(r):
        @pl.loop(0, _B)
        def _edge(i):
            for g in range(_F // 32):
                cf = 32 * g
                wf = 16 * g
                ws = _F // 2 + 16 * g
                wdf = _w16(g2[r], i, wf)
                wsf = _w16(g2[r], _B + i, wf)
                wef = _w16(eav[r], i, wf)
                wds = _w16(g2[r], i, ws)
                wss = _w16(g2[r], _B + i, ws)
                wes = _w16(eav[r], i, ws)
                for h, ext in enumerate((_lo_f32, _hi_f32)):
                    zf = ext(wdf) + ext(wsf) + ext(wef)
                    zs = ext(wds) + ext(wss) + ext(wes)
                    gate = 1.0 / (1.0 + jnp.exp(-zf))
                    msg[r][i, pl.ds(cf + 16 * h, 16)] = gate * _softplus(zs)

    fire_idx(0, 0)
    fire_idx(1, 1)

    @pl.loop(0, _ZROWS)
    def _zero(i):
        for j in range(_F // 16):
            zbuf[i, pl.ds(j * 16, 16)] = jnp.zeros((16,), jnp.float32)

    for t in range(_ROWS_W // _ZROWS):
        pltpu.sync_copy(zbuf, agg.at[pl.ds(s * _ROWS_W + t * _ZROWS, _ZROWS)])

    @pl.when(s == 0)
    def _zero_tail():
        pltpu.sync_copy(zbuf.at[pl.ds(0, _TAIL)],
                        agg.at[pl.ds(_NS * _ROWS_W, _TAIL)])

    wait_idx(0, 0)
    fire_gather(0, 0)
    plsc.subcore_barrier()

    wait_idx(1, 1)
    fire_gather(1, 1)
    wait_gather(0, 0)
    copy_dsts(0)
    fire_idx(2, 0)
    compute(0)
    fire_scatter(0)

    wait_idx(2, 0)
    fire_gather(2, 0)
    wait_gather(1, 1)
    copy_dsts(1)
    fire_idx(3, 1)
    compute(1)
    fire_scatter(1)

    @pl.loop(0, (_CHUNKS - 4) // 2)
    def _pair(m):
        k = 2 * m + 2
        for p in range(2):
            r = p
            rn = 1 - p
            wait_idx(k + p + 1, rn)
            fire_gather(k + p + 1, rn)
            wait_gather(k + p, r)
            wait_scatter(r)
            copy_dsts(r)
            fire_idx(k + p + 2, r)
            compute(r)
            fire_scatter(r)

    wait_idx(_CHUNKS - 1, 1)
    fire_gather(_CHUNKS - 1, 1)
    wait_gather(_CHUNKS - 2, 0)
    wait_scatter(0)
    copy_dsts(0)
    compute(0)
    fire_scatter(0)

    wait_gather(_CHUNKS - 1, 1)
    wait_scatter(1)
    copy_dsts(1)
    compute(1)
    fire_scatter(1)

    wait_scatter(0)
    wait_scatter(1)
    plsc.subcore_barrier()

    pltpu.sync_copy(agg.at[pl.ds(s * _ROWS_W, _ROWS_W)],
                    out_hbm.at[c, pl.ds(s * _ROWS_W, _ROWS_W)])

    @pl.when(s == 0)
    def _write_tail():
        pltpu.sync_copy(agg.at[pl.ds(_NS * _ROWS_W, _TAIL)],
                        out_hbm.at[c, pl.ds(_NS * _ROWS_W, _TAIL)])


def _sc_edge(T2, EA, ei):
    mesh = plsc.VectorSubcoreMesh(
        core_axis_name="c", subcore_axis_name="s",
        num_cores=_NC, num_subcores=_NS)
    cp = pltpu.CompilerParams()
    if "needs_layout_passes" in pltpu.CompilerParams.__dataclass_fields__:
        cp = dataclasses.replace(cp, needs_layout_passes=False)
    fn = pl.kernel(
        _sc_edge_body,
        out_type=jax.ShapeDtypeStruct((_NC, _N, _F), jnp.float32),
        mesh=mesh,
        compiler_params=cp,
        scratch_types=[
            [pltpu.VMEM((2 * _B + 16,), jnp.int32)] * 2,
            [pltpu.VMEM((_B,), jnp.int32)] * 2,
            [pltpu.VMEM((2 * _B, _F), jnp.int32)] * 2,
            [pltpu.VMEM((_B, _F), jnp.int32)] * 2,
            [pltpu.VMEM((_B, _F), jnp.float32)] * 2,
            pltpu.VMEM((_ZROWS, _F), jnp.float32),
            pltpu.VMEM_SHARED((_N, _F), jnp.float32),
            [pltpu.SemaphoreType.DMA] * 2,
            [pltpu.SemaphoreType.DMA] * 2,
            [pltpu.SemaphoreType.DMA] * 2,
            [pltpu.SemaphoreType.DMA] * 2,
        ],
    )
    return fn(T2, EA, ei)


def kernel(x, edge_index, edge_attr, batch, Wf0, bf0, Ws0, bs0,
           Wf1, bf1, Ws1, bs1, gamma0, beta0, gamma1, beta1, Wfc, bfc):
    ei = edge_index.astype(jnp.int32).reshape(2 * _E)
    seg = batch.astype(jnp.int32)

    lop = jnp.array(_LOPERM, dtype=jnp.int32)
    hip = jnp.array(_HIPERM, dtype=jnp.int32)

    def split(Wf, Ws, bf, bs):
        W = jnp.concatenate([Wf, Ws], axis=1)
        b = jnp.concatenate([bf, bs])
        Wd, Wsrc, We = W[:_F], W[_F:2 * _F], W[2 * _F:]
        return ((Wd[:, lop], Wd[:, hip]), (Wsrc[:, lop], Wsrc[:, hip]),
                (We[:, lop], We[:, hip]), (b[lop], b[hip]))

    Wd0, Wsrc0, We0, b0 = split(Wf0, Ws0, bf0, bs0)
    Wd1, Wsrc1, We1, b1 = split(Wf1, Ws1, bf1, bs1)

    EA0 = _edge_terms(edge_attr, We0[0], b0[0], We0[1], b0[1])
    EA1 = _edge_terms(edge_attr, We1[0], b1[0], We1[1], b1[1])

    T20 = _tables(x, jnp.stack([Wd0[0], Wsrc0[0]]),
                  jnp.stack([Wd0[1], Wsrc0[1]]))
    parts0 = _sc_edge(T20, EA0, ei)

    u0, s0, q0 = _stats(x, parts0)
    h1, T21 = _mid(u0, s0, q0, gamma0, beta0,
                   jnp.stack([Wd1[0], Wsrc1[0]]), jnp.stack([Wd1[1], Wsrc1[1]]))
    parts1 = _sc_edge(T21, EA1, ei)

    u1, s1, q1 = _stats(h1, parts1)
    return _final(u1, s1, q1, gamma1, beta1, seg, Wfc, bfc)


# speedup vs baseline: 6.7778x; 1.0809x over previous
import dataclasses
import functools

import jax
import jax.numpy as jnp
from jax import lax
from jax.experimental import pallas as pl
from jax.experimental.pallas import tpu as pltpu
from jax.experimental.pallas import tpu_sc as plsc

_N = 10000
_E = 320000
_F = 128
_D = 16
_G = 64
_C = 16
_EPS = 1e-5

_NC = 2
_NS = 16
_NW = _NC * _NS
_PER_W = _E // _NW
_B = 40
_CHUNKS = _PER_W // _B
_ROWS_W = 624
_TAIL = _N - _NS * _ROWS_W
_ZROWS = 48

_Q = (0.9999999705990521, -0.4999950177195202, 0.3331927456296667,
      -0.24844387923813097, 0.19111483813148478, -0.1367485547842574,
      0.07836244160823844, -0.029588863662033647, 0.0052535214410623764)

_PREC = lax.Precision.HIGHEST


def _dot(a, b):
    return jnp.dot(a, b, preferred_element_type=jnp.float32, precision=_PREC)


_LOPERM = tuple(32 * g + t for g in range(8) for t in range(16))
_HIPERM = tuple(32 * g + 16 + t for g in range(8) for t in range(16))


def _pack2(lo, hi):
    lob = lax.bitcast_convert_type(
        lo.astype(jnp.bfloat16).astype(jnp.float32), jnp.int32)
    hib = lax.bitcast_convert_type(
        hi.astype(jnp.bfloat16).astype(jnp.float32), jnp.int32)
    return jnp.bitwise_or(jnp.bitwise_and(hib, jnp.int32(-65536)),
                          lax.shift_right_logical(lob, 16))


def _edge_terms_body(ea_ref, wlo_ref, blo_ref, whi_ref, bhi_ref, o_ref):
    dn = (((0,), (0,)), ((), ()))
    ea_t = ea_ref[...]
    lo = lax.dot_general(ea_t, wlo_ref[...], dn,
                         preferred_element_type=jnp.float32)
    hi = lax.dot_general(ea_t, whi_ref[...], dn,
                         preferred_element_type=jnp.float32)
    o_ref[...] = _pack2(lo + blo_ref[...], hi + bhi_ref[...])


def _edge_terms(edge_attr_t, Wlo, blo, Whi, bhi):
    eb = 6400
    return pl.pallas_call(
        _edge_terms_body,
        grid=(_E // eb,),
        in_specs=[
            pl.BlockSpec((_D, eb), lambda i: (0, i)),
            pl.BlockSpec((_D, _F), lambda i: (0, 0)),
            pl.BlockSpec((1, _F), lambda i: (0, 0)),
            pl.BlockSpec((_D, _F), lambda i: (0, 0)),
            pl.BlockSpec((1, _F), lambda i: (0, 0)),
        ],
        out_specs=pl.BlockSpec((eb, _F), lambda i: (i, 0)),
        out_shape=jax.ShapeDtypeStruct((_E, _F), jnp.int32),
    )(edge_attr_t, Wlo, blo.reshape(1, -1), Whi, bhi.reshape(1, -1))


_TRB = 2000


def _tables_body(x_ref, wlo_ref, whi_ref, t2_ref):
    xv = x_ref[...]
    t2_ref[...] = _pack2(_dot(xv, wlo_ref[0]), _dot(xv, whi_ref[0]))


def _tables(h, W2lo, W2hi):
    nrb = _N // _TRB
    return pl.pallas_call(
        _tables_body,
        grid=(2, nrb),
        in_specs=[
            pl.BlockSpec((_TRB, _F), lambda t, i: (i, 0)),
            pl.BlockSpec((1, _F, _F), lambda t, i: (t, 0, 0)),
            pl.BlockSpec((1, _F, _F), lambda t, i: (t, 0, 0)),
        ],
        out_specs=pl.BlockSpec((_TRB, _F), lambda t, i: (t * nrb + i, 0)),
        out_shape=jax.ShapeDtypeStruct((2 * _N, _F), jnp.int32),
    )(h, W2lo, W2hi)


_RB = 2000
_NRB = _N // _RB


def _stats_body(x_ref, p_ref, u_ref, sum_ref, sq_ref):
    u = x_ref[...] + p_ref[0] + p_ref[1]
    u_ref[...] = u
    ps = jnp.sum(u, axis=0, keepdims=True)
    pq = jnp.sum(u * u, axis=0, keepdims=True)

    @pl.when(pl.program_id(0) == 0)
    def _init():
        sum_ref[...] = ps
        sq_ref[...] = pq

    @pl.when(pl.program_id(0) > 0)
    def _acc():
        sum_ref[...] += ps
        sq_ref[...] += pq


def _stats(x, parts):
    return pl.pallas_call(
        _stats_body,
        grid=(_NRB,),
        in_specs=[
            pl.BlockSpec((_RB, _F), lambda i: (i, 0)),
            pl.BlockSpec((2, _RB, _F), lambda i: (0, i, 0)),
        ],
        out_specs=[
            pl.BlockSpec((_RB, _F), lambda i: (i, 0)),
            pl.BlockSpec((1, _F), lambda i: (0, 0)),
            pl.BlockSpec((1, _F), lambda i: (0, 0)),
        ],
        out_shape=[
            jax.ShapeDtypeStruct((_N, _F), jnp.float32),
            jax.ShapeDtypeStruct((1, _F), jnp.float32),
            jax.ShapeDtypeStruct((1, _F), jnp.float32),
        ],
    )(x, parts)


def _bn_from_stats(u, s, q, gamma, beta):
    m = s / _N
    v = q / _N - m * m
    return (u - m) / jnp.sqrt(v + _EPS) * gamma + beta


def _mid_body(u_ref, s_ref, q_ref, g_ref, b_ref, wlo_ref, whi_ref,
              h_ref, t2_ref):
    h = _bn_from_stats(u_ref[...], s_ref[...], q_ref[...],
                       g_ref[...], b_ref[...])
    h_ref[...] = h
    t2_ref[...] = _pack2(_dot(h, wlo_ref[0]), _dot(h, whi_ref[0]))


def _mid(u, s, q, gamma, beta, W2lo, W2hi):
    nrb = _N // _RB
    return pl.pallas_call(
        _mid_body,
        grid=(2, nrb),
        in_specs=[
            pl.BlockSpec((_RB, _F), lambda t, i: (i, 0)),
            pl.BlockSpec((1, _F), lambda t, i: (0, 0)),
            pl.BlockSpec((1, _F), lambda t, i: (0, 0)),
            pl.BlockSpec((1, _F), lambda t, i: (0, 0)),
            pl.BlockSpec((1, _F), lambda t, i: (0, 0)),
            pl.BlockSpec((1, _F, _F), lambda t, i: (t, 0, 0)),
            pl.BlockSpec((1, _F, _F), lambda t, i: (t, 0, 0)),
        ],
        out_specs=[
            pl.BlockSpec((_RB, _F), lambda t, i: (i, 0)),
            pl.BlockSpec((_RB, _F), lambda t, i: (t * nrb + i, 0)),
        ],
        out_shape=[
            jax.ShapeDtypeStruct((_N, _F), jnp.float32),
            jax.ShapeDtypeStruct((2 * _N, _F), jnp.int32),
        ],
    )(u, s, q, gamma.reshape(1, -1), beta.reshape(1, -1), W2lo, W2hi)


def _final_body(u_ref, s_ref, q_ref, g_ref, b_ref, seg_ref, wfc_ref, bfc_ref,
                o_ref, sacc_ref, cacc_ref):
    h2 = _bn_from_stats(u_ref[...], s_ref[...], q_ref[...],
                        g_ref[...], b_ref[...])
    onehot = (seg_ref[...] == lax.broadcasted_iota(jnp.int32, (_RB, _G), 1)
              ).astype(jnp.float32)
    ps = lax.dot_general(onehot, h2, (((0,), (0,)), ((), ())),
                         precision=_PREC, preferred_element_type=jnp.float32)
    pc = jnp.sum(onehot, axis=0)[:, None]
    i = pl.program_id(0)

    @pl.when(i == 0)
    def _init():
        sacc_ref[...] = ps
        cacc_ref[...] = pc

    @pl.when(i > 0)
    def _acc():
        sacc_ref[...] += ps
        cacc_ref[...] += pc

    @pl.when(i == _NRB - 1)
    def _emit():
        pooled = sacc_ref[...] / jnp.clip(cacc_ref[...], 1.0, None)
        o_ref[...] = _dot(pooled, wfc_ref[...]) + bfc_ref[...]


def _final(u, s, q, gamma, beta, seg, Wfc, bfc):
    return pl.pallas_call(
        _final_body,
        grid=(_NRB,),
        in_specs=[
            pl.BlockSpec((_RB, _F), lambda i: (i, 0)),
            pl.BlockSpec((1, _F), lambda i: (0, 0)),
            pl.BlockSpec((1, _F), lambda i: (0, 0)),
            pl.BlockSpec((1, _F), lambda i: (0, 0)),
            pl.BlockSpec((1, _F), lambda i: (0, 0)),
            pl.BlockSpec((_RB, 1), lambda i: (i, 0)),
            pl.BlockSpec((_F, _C), lambda i: (0, 0)),
            pl.BlockSpec((1, _C), lambda i: (0, 0)),
        ],
        out_specs=pl.BlockSpec((_G, _C), lambda i: (0, 0)),
        out_shape=jax.ShapeDtypeStruct((_G, _C), jnp.float32),
        scratch_shapes=[
            pltpu.VMEM((_G, _F), jnp.float32),
            pltpu.VMEM((_G, 1), jnp.float32),
        ],
    )(u, s, q, gamma.reshape(1, -1), beta.reshape(1, -1),
      seg.reshape(_N, 1), Wfc, bfc.reshape(1, -1))


def _softplus(x):
    u = jnp.exp(-jnp.abs(x))
    q = jnp.float32(_Q[-1])
    for c in _Q[-2::-1]:
        q = q * u + jnp.float32(c)
    return jnp.maximum(x, 0.0) + q * u


def _lo_f32(w):
    return plsc.bitcast(jnp.left_shift(w, 16), jnp.float32)


def _hi_f32(w):
    return plsc.bitcast(jnp.bitwise_and(w, jnp.int32(-65536)), jnp.float32)


def _w16(ref, i, wcol):
    return ref[i, pl.ds(wcol, 16)]


def _sc_edge_body(t2_hbm, ea_hbm, ei_hbm, out_hbm,
                  idx, dsts, g2, eav, msg, zbuf, agg, semi, semg, seme, sems):
    c = lax.axis_index("c")
    s = lax.axis_index("s")
    base = (c * _NS + s) * _PER_W

    def fire_idx(k, r):
        off = base + k * _B
        pltpu.async_copy(ei_hbm.at[pl.ds(_E + off, _B)],
                         idx[r].at[pl.ds(0, _B)], semi[r])
        pltpu.async_copy(ei_hbm.at[pl.ds(off, _B)],
                         idx[r].at[pl.ds(_B, _B)], semi[r])

    def wait_idx(k, r):
        off = base + k * _B
        pltpu.make_async_copy(ei_hbm.at[pl.ds(_E + off, _B)],
                              idx[r].at[pl.ds(0, _B)], semi[r]).wait()
        pltpu.make_async_copy(ei_hbm.at[pl.ds(off, _B)],
                              idx[r].at[pl.ds(_B, _B)], semi[r]).wait()
        for o in (_B, _B + 16, _B + 32):
            idx[r][pl.ds(o, 16)] = idx[r][pl.ds(o, 16)] + _N

    def fire_gather(k, r):
        off = base + k * _B
        pltpu.async_copy(t2_hbm.at[idx[r].at[pl.ds(0, 2 * _B)]],
                         g2[r], semg[r])
        pltpu.async_copy(ea_hbm.at[pl.ds(off, _B)], eav[r], seme[r])

    def wait_gather(k, r):
        off = base + k * _B
        pltpu.make_async_copy(t2_hbm.at[idx[r].at[pl.ds(0, 2 * _B)]],
                              g2[r], semg[r]).wait()
        pltpu.make_async_copy(ea_hbm.at[pl.ds(off, _B)], eav[r],
                              seme[r]).wait()

    def fire_scatter(r):
        pltpu.async_copy(msg[r], agg.at[dsts[r]], sems[r], add=True)

    def wait_scatter(r):
        pltpu.make_async_copy(msg[r], agg.at[dsts[r]], sems[r]).wait()

    def copy_dsts(r):
        dsts[r][pl.ds(0, 16)] = idx[r][pl.ds(0, 16)]
        dsts[r][pl.ds(16, 16)] = idx[r][pl.ds(16, 16)]
        dsts[r][pl.ds(24, 16)] = idx[r][pl.ds(24, 16)]

    def compute(r):
        @pl.loop(0, _B)
        def _edge(i):
            for g in range(_F // 32):
                cf = 32 * g
                wf = 16 * g
                ws = _F // 2 + 16 * g
                wdf = _w16(g2[r], i, wf)
                wsf = _w16(g2[r], _B + i, wf)
                wef = _w16(eav[r], i, wf)
                wds = _w16(g2[r], i, ws)
                wss = _w16(g2[r], _B + i, ws)
                wes = _w16(eav[r], i, ws)
                for h, ext in enumerate((_lo_f32, _hi_f32)):
                    zf = ext(wdf) + ext(wsf) + ext(wef)
                    zs = ext(wds) + ext(wss) + ext(wes)
                    gate = 1.0 / (1.0 + jnp.exp(-zf))
                    msg[r][i, pl.ds(cf + 16 * h, 16)] = gate * _softplus(zs)

    fire_idx(0, 0)
    fire_idx(1, 1)

    @pl.loop(0, _ZROWS)
    def _zero(i):
        for j in range(_F // 16):
            zbuf[i, pl.ds(j * 16, 16)] = jnp.zeros((16,), jnp.float32)

    for t in range(_ROWS_W // _ZROWS):
        pltpu.sync_copy(zbuf, agg.at[pl.ds(s * _ROWS_W + t * _ZROWS, _ZROWS)])

    @pl.when(s == 0)
    def _zero_tail():
        pltpu.sync_copy(zbuf.at[pl.ds(0, _TAIL)],
                        agg.at[pl.ds(_NS * _ROWS_W, _TAIL)])

    wait_idx(0, 0)
    fire_gather(0, 0)
    plsc.subcore_barrier()

    wait_idx(1, 1)
    fire_gather(1, 1)
    wait_gather(0, 0)
    copy_dsts(0)
    fire_idx(2, 0)
    compute(0)
    fire_scatter(0)

    wait_idx(2, 0)
    fire_gather(2, 0)
    wait_gather(1, 1)
    copy_dsts(1)
    fire_idx(3, 1)
    compute(1)
    fire_scatter(1)

    @pl.loop(0, (_CHUNKS - 4) // 2)
    def _pair(m):
        k = 2 * m + 2
        for p in range(2):
            r = p
            rn = 1 - p
            wait_idx(k + p + 1, rn)
            fire_gather(k + p + 1, rn)
            wait_gather(k + p, r)
            wait_scatter(r)
            copy_dsts(r)
            fire_idx(k + p + 2, r)
            compute(r)
            fire_scatter(r)

    wait_idx(_CHUNKS - 1, 1)
    fire_gather(_CHUNKS - 1, 1)
    wait_gather(_CHUNKS - 2, 0)
    wait_scatter(0)
    copy_dsts(0)
    compute(0)
    fire_scatter(0)

    wait_gather(_CHUNKS - 1, 1)
    wait_scatter(1)
    copy_dsts(1)
    compute(1)
    fire_scatter(1)

    wait_scatter(0)
    wait_scatter(1)
    plsc.subcore_barrier()

    pltpu.sync_copy(agg.at[pl.ds(s * _ROWS_W, _ROWS_W)],
                    out_hbm.at[c, pl.ds(s * _ROWS_W, _ROWS_W)])

    @pl.when(s == 0)
    def _write_tail():
        pltpu.sync_copy(agg.at[pl.ds(_NS * _ROWS_W, _TAIL)],
                        out_hbm.at[c, pl.ds(_NS * _ROWS_W, _TAIL)])


def _sc_edge(T2, EA, ei):
    mesh = plsc.VectorSubcoreMesh(
        core_axis_name="c", subcore_axis_name="s",
        num_cores=_NC, num_subcores=_NS)
    cp = pltpu.CompilerParams()
    if "needs_layout_passes" in pltpu.CompilerParams.__dataclass_fields__:
        cp = dataclasses.replace(cp, needs_layout_passes=False)
    fn = pl.kernel(
        _sc_edge_body,
        out_type=jax.ShapeDtypeStruct((_NC, _N, _F), jnp.float32),
        mesh=mesh,
        compiler_params=cp,
        scratch_types=[
            [pltpu.VMEM((2 * _B + 16,), jnp.int32)] * 2,
            [pltpu.VMEM((_B,), jnp.int32)] * 2,
            [pltpu.VMEM((2 * _B, _F), jnp.int32)] * 2,
            [pltpu.VMEM((_B, _F), jnp.int32)] * 2,
            [pltpu.VMEM((_B, _F), jnp.float32)] * 2,
            pltpu.VMEM((_ZROWS, _F), jnp.float32),
            pltpu.VMEM_SHARED((_N, _F), jnp.float32),
            [pltpu.SemaphoreType.DMA] * 2,
            [pltpu.SemaphoreType.DMA] * 2,
            [pltpu.SemaphoreType.DMA] * 2,
            [pltpu.SemaphoreType.DMA] * 2,
        ],
    )
    return fn(T2, EA, ei)


def kernel(x, edge_index, edge_attr, batch, Wf0, bf0, Ws0, bs0,
           Wf1, bf1, Ws1, bs1, gamma0, beta0, gamma1, beta1, Wfc, bfc):
    ei = edge_index.astype(jnp.int32).reshape(2 * _E)
    seg = batch.astype(jnp.int32)

    lop = jnp.array(_LOPERM, dtype=jnp.int32)
    hip = jnp.array(_HIPERM, dtype=jnp.int32)

    def split(Wf, Ws, bf, bs):
        W = jnp.concatenate([Wf, Ws], axis=1)
        b = jnp.concatenate([bf, bs])
        Wd, Wsrc, We = W[:_F], W[_F:2 * _F], W[2 * _F:]
        return ((Wd[:, lop], Wd[:, hip]), (Wsrc[:, lop], Wsrc[:, hip]),
                (We[:, lop], We[:, hip]), (b[lop], b[hip]))

    Wd0, Wsrc0, We0, b0 = split(Wf0, Ws0, bf0, bs0)
    Wd1, Wsrc1, We1, b1 = split(Wf1, Ws1, bf1, bs1)

    edge_attr_t = edge_attr.T
    EA0 = _edge_terms(edge_attr_t, We0[0], b0[0], We0[1], b0[1])
    EA1 = _edge_terms(edge_attr_t, We1[0], b1[0], We1[1], b1[1])

    T20 = _tables(x, jnp.stack([Wd0[0], Wsrc0[0]]),
                  jnp.stack([Wd0[1], Wsrc0[1]]))
    parts0 = _sc_edge(T20, EA0, ei)

    u0, s0, q0 = _stats(x, parts0)
    h1, T21 = _mid(u0, s0, q0, gamma0, beta0,
                   jnp.stack([Wd1[0], Wsrc1[0]]), jnp.stack([Wd1[1], Wsrc1[1]]))
    parts1 = _sc_edge(T21, EA1, ei)

    u1, s1, q1 = _stats(h1, parts1)
    return _final(u1, s1, q1, gamma1, beta1, seg, Wfc, bfc)
